# Initial kernel scaffold; baseline (speedup 1.0000x reference)
import jax
import jax.numpy as jnp
from jax.experimental import pallas as pl


def kernel(x_feat, node_ent, edge_index, edge_type, edge_ts, src, dst, q_rel, q_ts, ptr, node_emb_w, node_emb_b, rel_emb_table, ent_emb_table, time_emb, fc_w, fc_b, rt_w, rt_b):
    raise NotImplementedError("write your pallas kernel here")


# speedup vs baseline: 6.0915x; 6.0915x over previous
import functools

import jax
import jax.numpy as jnp
from jax import lax
from jax.experimental import pallas as pl
from jax.experimental.pallas import tpu as pltpu
from jax.experimental.pallas import tpu_sc as plsc

N = 10000
NP = 10240
E = 320000
H = 64
F = 2 * H
NREL = 200
NTS = 365
NCOMBO = NREL * NTS

NC = 2
NS = 16
NWORK = NC * NS
CH = 128
NCHUNK = 79
PER_W = CH * NCHUNK
EP = PER_W * NWORK
TRASH = NP - 1
RPT = NP // NS
ACCW = F + 16

_mesh = plsc.VectorSubcoreMesh(
    core_axis_name="c", subcore_axis_name="s", num_cores=NC, num_subcores=NS)


def _lrelu(v):
    return jnp.maximum(v, 0.2 * v)


@functools.partial(
    pl.kernel,
    out_type=[
        jax.ShapeDtypeStruct((EP,), jnp.int32),
        jax.ShapeDtypeStruct((NP, F), jnp.float32),
    ],
    mesh=_mesh,
    scratch_types=[
        pltpu.VMEM((CH,), jnp.int32),
        pltpu.VMEM((CH,), jnp.int32),
        pltpu.VMEM((CH,), jnp.int32),
        pltpu.VMEM((80,), jnp.int32),
        pltpu.VMEM((80, F), jnp.float32),
        pltpu.SemaphoreType.DMA,
    ],
    compiler_params=pltpu.CompilerParams(needs_layout_passes=False),
)
def _sc_prep(et_hbm, ets_hbm, ne_hbm, ent_tab_hbm, combo_out, ent_out,
             et_v, ets_v, cb_v, ni_v, er_v, sem):
    cid = lax.axis_index("c")
    sid = lax.axis_index("s")
    wid = sid * NC + cid
    base = wid * PER_W

    def chunk(i):
        off = base + i * CH
        pltpu.sync_copy(et_hbm.at[pl.ds(off, CH)], et_v)
        pltpu.sync_copy(ets_hbm.at[pl.ds(off, CH)], ets_v)
        for g in range(CH // 16):
            sl = pl.ds(g * 16, 16)
            cb_v[sl] = et_v[sl] * NTS + ets_v[sl]
        pltpu.sync_copy(cb_v, combo_out.at[pl.ds(off, CH)])

    pl.loop(0, NCHUNK)(chunk)

    def nchunk(i):
        off = wid * (NP // NWORK) + i * 80
        pltpu.sync_copy(ne_hbm.at[pl.ds(off, 80)], ni_v)
        pltpu.async_copy(ent_tab_hbm.at[ni_v], er_v, sem).wait()
        pltpu.sync_copy(er_v, ent_out.at[pl.ds(off, 80)])

    pl.loop(0, (NP // NWORK) // 80)(nchunk)


def _combo_body(rel_ref, time_ref, a1_ref, a2_ref, rtb_ref, wr_ref, fcb_ref,
                out_ref):
    relc = jnp.dot(rel_ref[...].reshape(1, H), a1_ref[...],
                   preferred_element_type=jnp.float32)
    z = jnp.dot(time_ref[...], a2_ref[...],
                preferred_element_type=jnp.float32)
    z = z + relc + rtb_ref[...]
    rt = _lrelu(z)
    out = jnp.dot(rt, wr_ref[...],
                  preferred_element_type=jnp.float32) + fcb_ref[...]
    out_ref[...] = out.reshape(1, NTS, F)


def _tc_combo(rel_emb, time_emb, a1, a2, rtb, wr, fcb):
    return pl.pallas_call(
        _combo_body,
        grid=(NREL,),
        in_specs=[
            pl.BlockSpec((1, 1, H), lambda r: (r, 0, 0)),
            pl.BlockSpec((NTS, H), lambda r: (0, 0)),
            pl.BlockSpec((H, F), lambda r: (0, 0)),
            pl.BlockSpec((H, F), lambda r: (0, 0)),
            pl.BlockSpec((1, F), lambda r: (0, 0)),
            pl.BlockSpec((F, F), lambda r: (0, 0)),
            pl.BlockSpec((1, F), lambda r: (0, 0)),
        ],
        out_specs=pl.BlockSpec((1, NTS, F), lambda r: (r, 0, 0)),
        out_shape=jax.ShapeDtypeStruct((NREL, NTS, F), jnp.float32),
    )(rel_emb.reshape(NREL, 1, H), time_emb, a1, a2, rtb, wr, fcb)


_NBLK = 256
_NGRID = NP // _NBLK


def _init_body(qs_ref, qo_ref, xf_ref, ent_ref, wn_ref, nb_ref, ws_ref,
               wd_ref, x_ref, pxs_ref, pxd_ref, t0_ref, tfin_ref):
    b = pl.program_id(0)
    h = _lrelu(jnp.dot(xf_ref[...], wn_ref[...],
                       preferred_element_type=jnp.float32) + nb_ref[...])
    x = jnp.concatenate([h, ent_ref[:, pl.ds(0, H)]], axis=1)
    x_ref[...] = x
    pxs_ref[...] = jnp.dot(x, ws_ref[...], preferred_element_type=jnp.float32)
    pxd_ref[...] = jnp.dot(x, wd_ref[...], preferred_element_type=jnp.float32)
    rows = _NBLK // F
    ids = (b * _NBLK
           + lax.broadcasted_iota(jnp.int32, (rows, F), 0) * F
           + lax.broadcasted_iota(jnp.int32, (rows, F), 1))
    act = jnp.zeros((rows, F), jnp.int32)
    iso = jnp.zeros((rows, F), jnp.int32)
    for j in range(4):
        act = jnp.maximum(act, (ids == qs_ref[j]).astype(jnp.int32))
        iso = jnp.maximum(iso, (ids == qo_ref[j]).astype(jnp.int32))
    t0_ref[...] = (2 * act + (1 - iso)).reshape(1, rows, F)
    tfin_ref[...] = (2 + iso).reshape(1, rows, F)


def _tc_node_init(xf, ent_rows, wn, nb, ws, wd, q_s, q_o):
    rows = _NBLK // F
    out2 = jax.ShapeDtypeStruct((_NGRID, rows, F), jnp.int32)
    outs = [jax.ShapeDtypeStruct((NP, F), jnp.float32)] * 3 + [out2] * 2
    big = pl.BlockSpec((_NBLK, F), lambda b: (b, 0))
    tbl = pl.BlockSpec((1, rows, F), lambda b: (b, 0, 0))
    return pl.pallas_call(
        _init_body,
        grid=(_NGRID,),
        in_specs=[
            pl.BlockSpec(memory_space=pltpu.SMEM),
            pl.BlockSpec(memory_space=pltpu.SMEM),
            big,
            big,
            pl.BlockSpec((F, H), lambda b: (0, 0)),
            pl.BlockSpec((1, H), lambda b: (0, 0)),
            pl.BlockSpec((F, F), lambda b: (0, 0)),
            pl.BlockSpec((F, F), lambda b: (0, 0)),
        ],
        out_specs=[big, big, big, tbl, tbl],
        out_shape=outs,
    )(q_s, q_o, xf, ent_rows, wn, nb, ws, wd)


_CPR = NP // 8
_CPT = _CPR // NS
ECH = 32
ENCH = PER_W // ECH


@functools.partial(
    pl.kernel,
    out_type=[
        jax.ShapeDtypeStruct((NC, NP, F), jnp.float32),
        jax.ShapeDtypeStruct((NC, NP), jnp.float32),
        jax.ShapeDtypeStruct((NC, _CPR, F), jnp.float32),
    ],
    mesh=_mesh,
    scratch_types=[
        pltpu.VMEM((ECH,), jnp.int32),
        pltpu.VMEM((ECH,), jnp.int32),
        pltpu.VMEM((ECH,), jnp.int32),
        pltpu.VMEM((ECH,), jnp.int32),
        pltpu.VMEM((ECH,), jnp.int32),
        pltpu.VMEM((ECH,), jnp.int32),
        pltpu.VMEM((ECH,), jnp.float32),
        pltpu.VMEM((ECH, F), jnp.float32),
        pltpu.VMEM((ECH, F), jnp.float32),
        pltpu.VMEM((ECH, F), jnp.float32),
        pltpu.VMEM((ECH, F), jnp.float32),
        pltpu.VMEM((ECH, F), jnp.float32),
        pltpu.VMEM((RPT,), jnp.float32),
        pltpu.VMEM((NP,), jnp.int32),
        pltpu.VMEM_SHARED((NP, F), jnp.float32),
        pltpu.VMEM_SHARED((_CPR, F), jnp.float32),
        pltpu.SemaphoreType.DMA,
        pltpu.SemaphoreType.DMA,
    ],
    compiler_params=pltpu.CompilerParams(needs_layout_passes=False),
)
def _sc_edge(src_hbm, dst_hbm, cb_hbm, ctab_hbm, pxs_hbm, pxd_hbm, t_hbm,
             acc_out, cnt_out, cntp_out,
             src_v, dst_v, si_v, si2_v, dm_v, ci_v, w_v, a_v, b_v, c_v,
             o_v, o2_v, ce_v, t_v, accm_sh, accc_sh, sem1, sem2):
    cid = lax.axis_index("c")
    sid = lax.axis_index("s")
    wid = sid * NC + cid
    base = wid * PER_W
    tb = sid * RPT

    pltpu.sync_copy(t_hbm, t_v)

    zeros16 = jnp.zeros((16,), jnp.float32)

    def zrow(r):
        for s in range(F // 16):
            o_v[r, pl.ds(s * 16, 16)] = zeros16

    pl.loop(0, ECH)(zrow)

    def zchunk(i):
        pltpu.sync_copy(o_v, accm_sh.at[pl.ds(tb + i * ECH, ECH)])

    pl.loop(0, RPT // ECH)(zchunk)

    def zchunk2(i):
        pltpu.sync_copy(o_v.at[pl.ds(0, 16)],
                        accc_sh.at[pl.ds(sid * _CPT + i * 16, 16)])

    pl.loop(0, _CPT // 16)(zchunk2)
    plsc.subcore_barrier()

    def chunk(i):
        off = base + i * ECH
        pltpu.sync_copy(src_hbm.at[pl.ds(off, ECH)], src_v)
        pltpu.sync_copy(dst_hbm.at[pl.ds(off, ECH)], dst_v)
        pltpu.sync_copy(cb_hbm.at[pl.ds(off, ECH)], ci_v)

        def wgrp(mw, g):
            sl = pl.ds(g * 16, 16)
            sv = src_v[sl]
            dv = dst_v[sl]
            ts = plsc.load_gather(t_v, [sv])
            td = plsc.load_gather(t_v, [dv])
            ok = jnp.logical_and(ts >= 2, lax.bitwise_and(td, 1) == 1)
            w = jnp.where(ok, 1.0, 0.0)
            w_v[sl] = w
            si = jnp.where(ok, dv, TRASH)
            si_v[sl] = si
            si2_v[sl] = lax.shift_right_logical(si, 3)
            dm_v[sl] = lax.bitwise_and(dv, 7)
            return jnp.maximum(mw, jnp.max(w))

        mw = 0.0
        for g in range(ECH // 16):
            mw = wgrp(mw, g)

        @pl.when(mw > 0.0)
        def _():
            d1 = pltpu.async_copy(pxs_hbm.at[src_v], a_v, sem1)
            d2 = pltpu.async_copy(pxd_hbm.at[dst_v], b_v, sem1)
            d3 = pltpu.async_copy(ctab_hbm.at[ci_v], c_v, sem2)
            d1.wait()
            d2.wait()
            d3.wait()

            def edge(j):
                jb = jnp.full((16,), j, jnp.int32)
                wb = plsc.load_gather(w_v, [jb])

                @pl.when(jnp.max(wb) > 0.0)
                def _():
                    for s in range(F // 16):
                        sl = pl.ds(s * 16, 16)
                        v = a_v[j, sl] + b_v[j, sl] + c_v[j, sl]
                        o_v[j, sl] = _lrelu(v) * wb
                    dmb = plsc.load_gather(dm_v, [jb])
                    for k in range(8):
                        o2_v[j, pl.ds(k * 16, 16)] = jnp.where(
                            dmb == k, wb, 0.0)

            pl.loop(0, ECH)(edge)
            pltpu.sync_copy(o_v, accm_sh.at[si_v], add=True)
            pltpu.sync_copy(o2_v, accc_sh.at[si2_v], add=True)

    pl.loop(0, ENCH)(chunk)
    plsc.subcore_barrier()

    pltpu.sync_copy(accm_sh.at[pl.ds(tb, RPT)],
                    acc_out.at[cid, pl.ds(tb, RPT)])

    def cstage(i):
        pltpu.sync_copy(accc_sh.at[pl.ds(sid * _CPT + i * 16, 16)],
                        o_v.at[pl.ds(0, 16)])
        it = lax.iota(jnp.int32, 16)

        for g in range(8):
            ridx = lax.shift_right_logical(it + g * 16, 3)
            cidx = lax.bitwise_and(it, 7) * 16
            ce_v[pl.ds(i * CH + g * 16, 16)] = plsc.load_gather(
                o_v, [ridx, cidx])

        def crow(r):
            rb = jnp.full((16,), r, jnp.int32)
            for k in range(8):
                o2_v[r, pl.ds(k * 16, 16)] = plsc.load_gather(
                    o_v, [rb, jnp.full((16,), k * 16, jnp.int32)])

        pl.loop(0, 16)(crow)
        pltpu.sync_copy(o2_v.at[pl.ds(0, 16)],
                        cntp_out.at[cid, pl.ds(sid * _CPT + i * 16, 16)])

    pl.loop(0, _CPT // 16)(cstage)
    pltpu.sync_copy(ce_v, cnt_out.at[cid, pl.ds(tb, RPT)])


def _round_body(x_ref, m0_ref, m1_ref, c0_ref, c1_ref, ws_ref, wd_ref,
                xn_ref, pxs_ref, pxd_ref):
    s = m0_ref[...] + m1_ref[...]
    c = c0_ref[...] + c1_ref[...]
    cnt = jnp.max(c, axis=1, keepdims=True)
    xn = x_ref[...] + s / jnp.maximum(cnt, 1.0)
    xn_ref[...] = xn
    pxs_ref[...] = jnp.dot(xn, ws_ref[...], preferred_element_type=jnp.float32)
    pxd_ref[...] = jnp.dot(xn, wd_ref[...], preferred_element_type=jnp.float32)


def _tc_round(x, m0, m1, c0, c1, ws, wd):
    big = pl.BlockSpec((_NBLK, F), lambda b: (b, 0))
    csp = pl.BlockSpec((_NBLK, 16), lambda b: (b, 0))
    return pl.pallas_call(
        _round_body,
        grid=(_NGRID,),
        in_specs=[
            big, big, big, csp, csp,
            pl.BlockSpec((F, F), lambda b: (0, 0)),
            pl.BlockSpec((F, F), lambda b: (0, 0)),
        ],
        out_specs=[big, big, big],
        out_shape=[jax.ShapeDtypeStruct((NP, F), jnp.float32)] * 3,
    )(x, m0, m1, c0, c1, ws, wd)


def _query_body(qr_ref, qt_ref, rel_ref, time_ref, a1_ref, a2_ref, rtb_ref,
                out_ref):
    r0 = lax.broadcasted_iota(jnp.int32, (8, NREL), 0)
    ir = lax.broadcasted_iota(jnp.int32, (8, NREL), 1)
    qv = jnp.full((8, NREL), qr_ref[3], jnp.int32)
    for j in range(3):
        qv = jnp.where(r0 == j, qr_ref[j], qv)
    oh_r = (ir == qv).astype(jnp.float32)
    t0 = lax.broadcasted_iota(jnp.int32, (8, NTS), 0)
    it = lax.broadcasted_iota(jnp.int32, (8, NTS), 1)
    tv = jnp.full((8, NTS), qt_ref[3], jnp.int32)
    for j in range(3):
        tv = jnp.where(t0 == j, qt_ref[j], tv)
    oh_t = (it == tv).astype(jnp.float32)
    rel_e = jnp.dot(oh_r, rel_ref[...], preferred_element_type=jnp.float32)
    time_e = jnp.dot(oh_t, time_ref[...], preferred_element_type=jnp.float32)
    z = (jnp.dot(rel_e, a1_ref[...], preferred_element_type=jnp.float32)
         + jnp.dot(time_e, a2_ref[...], preferred_element_type=jnp.float32)
         + rtb_ref[...])
    out_ref[...] = _lrelu(z)


def _tc_query(rel_emb, time_emb, a1, a2, rtb, q_rel, q_ts):
    return pl.pallas_call(
        _query_body,
        grid=(1,),
        in_specs=[
            pl.BlockSpec(memory_space=pltpu.SMEM),
            pl.BlockSpec(memory_space=pltpu.SMEM),
            pl.BlockSpec((NREL, H), lambda b: (0, 0)),
            pl.BlockSpec((NTS, H), lambda b: (0, 0)),
            pl.BlockSpec((H, F), lambda b: (0, 0)),
            pl.BlockSpec((H, F), lambda b: (0, 0)),
            pl.BlockSpec((1, F), lambda b: (0, 0)),
        ],
        out_specs=pl.BlockSpec((8, F), lambda b: (0, 0)),
        out_shape=jax.ShapeDtypeStruct((8, F), jnp.float32),
    )(q_rel, q_ts, rel_emb, time_emb, a1, a2, rtb)


def kernel(x_feat, node_ent, edge_index, edge_type, edge_ts, src, dst,
           q_rel, q_ts, ptr, node_emb_w, node_emb_b, rel_emb_table,
           ent_emb_table, time_emb, fc_w, fc_b, rt_w, rt_b):
    i32 = jnp.int32
    f32 = jnp.float32
    q_s = (src + ptr[:-1]).astype(i32)
    q_o = (dst + ptr[:-1]).astype(i32)

    pad_e = EP - E
    srcp = jnp.concatenate([edge_index[0].astype(i32),
                            jnp.zeros((pad_e,), i32)])
    dstp = jnp.concatenate([edge_index[1].astype(i32),
                            jnp.full((pad_e,), TRASH, i32)])
    etp = jnp.concatenate([edge_type.astype(i32), jnp.zeros((pad_e,), i32)])
    etsp = jnp.concatenate([edge_ts.astype(i32), jnp.zeros((pad_e,), i32)])
    nep = jnp.concatenate([node_ent.astype(i32), jnp.zeros((NP - N,), i32)])
    xfp = jnp.concatenate([x_feat, jnp.zeros((NP - N, x_feat.shape[1]), f32)])

    entp = jnp.concatenate(
        [ent_emb_table, jnp.zeros((N, F - H), f32)], axis=1)
    combo, ent_rows = _sc_prep(etp, etsp, nep, entp)

    a1 = rt_w[:, :H].T
    a2 = rt_w[:, H:].T
    ws = fc_w[:, :F].T
    wr = fc_w[:, F:2 * F].T
    wd = fc_w[:, 2 * F:].T
    rtb2 = rt_b.reshape(1, F)
    fcb2 = fc_b.reshape(1, F)
    nb2 = node_emb_b.reshape(1, H)

    ctab = _tc_combo(rel_emb_table, time_emb, a1, a2, rtb2, wr,
                     fcb2).reshape(NCOMBO, F)

    x, pxs, pxd, t0, tfin = _tc_node_init(
        xfp, ent_rows, node_emb_w.T, nb2, ws, wd, q_s, q_o)

    t = t0.reshape(NP)
    tnot_bit = lax.bitwise_and(t, 1)
    for _ in range(3):
        accs, cnts, cntp = _sc_edge(srcp, dstp, combo, ctab, pxs, pxd, t)
        cp = cntp.reshape(NC, NP, 16)
        x, pxs, pxd = _tc_round(x, accs[0], accs[1], cp[0], cp[1], ws, wd)
        t = 2 * (cnts[0] + cnts[1] > 0.0).astype(i32) + tnot_bit

    accs, _, cntp = _sc_edge(srcp, dstp, combo, ctab, pxs, pxd,
                             tfin.reshape(NP))
    cp = cntp.reshape(NC, NP, 16)
    x, _, _ = _tc_round(x, accs[0], accs[1], cp[0], cp[1], ws, wd)

    qrt = _tc_query(rel_emb_table, time_emb, a1, a2, rtb2, q_rel.astype(i32),
                    q_ts.astype(i32))
    return (x[:N], qrt[:4])


# speedup vs baseline: 7.7144x; 1.2664x over previous
import functools

import jax
import jax.numpy as jnp
from jax import lax
from jax.experimental import pallas as pl
from jax.experimental.pallas import tpu as pltpu
from jax.experimental.pallas import tpu_sc as plsc

N = 10000
NP = 10240
E = 320000
H = 64
F = 2 * H
NREL = 200
NTS = 365
NCOMBO = NREL * NTS

NC = 2
NS = 16
NWORK = NC * NS
CH = 128
NCHUNK = 80
PER_W = CH * NCHUNK
EP = PER_W * NWORK
TRASH = NP - 1
RPT = NP // NS
ACCW = F + 16

_mesh = plsc.VectorSubcoreMesh(
    core_axis_name="c", subcore_axis_name="s", num_cores=NC, num_subcores=NS)


def _lrelu(v):
    return jnp.maximum(v, 0.2 * v)


@functools.partial(
    pl.kernel,
    out_type=[
        jax.ShapeDtypeStruct((EP,), jnp.int32),
        jax.ShapeDtypeStruct((NP, F), jnp.float32),
    ],
    mesh=_mesh,
    scratch_types=[
        pltpu.VMEM((CH,), jnp.int32),
        pltpu.VMEM((CH,), jnp.int32),
        pltpu.VMEM((CH,), jnp.int32),
        pltpu.VMEM((80,), jnp.int32),
        pltpu.VMEM((80, F), jnp.float32),
        pltpu.SemaphoreType.DMA,
    ],
    compiler_params=pltpu.CompilerParams(needs_layout_passes=False),
)
def _sc_prep(et_hbm, ets_hbm, ne_hbm, ent_tab_hbm, combo_out, ent_out,
             et_v, ets_v, cb_v, ni_v, er_v, sem):
    cid = lax.axis_index("c")
    sid = lax.axis_index("s")
    wid = sid * NC + cid
    base = wid * PER_W

    def chunk(i):
        off = base + i * CH
        pltpu.sync_copy(et_hbm.at[pl.ds(off, CH)], et_v)
        pltpu.sync_copy(ets_hbm.at[pl.ds(off, CH)], ets_v)
        for g in range(CH // 16):
            sl = pl.ds(g * 16, 16)
            cb_v[sl] = et_v[sl] * NTS + ets_v[sl]
        pltpu.sync_copy(cb_v, combo_out.at[pl.ds(off, CH)])

    pl.loop(0, NCHUNK)(chunk)

    def nchunk(i):
        off = wid * (NP // NWORK) + i * 80
        pltpu.sync_copy(ne_hbm.at[pl.ds(off, 80)], ni_v)
        pltpu.async_copy(ent_tab_hbm.at[ni_v], er_v, sem).wait()
        pltpu.sync_copy(er_v, ent_out.at[pl.ds(off, 80)])

    pl.loop(0, (NP // NWORK) // 80)(nchunk)


def _combo_body(rel_ref, time_ref, a1_ref, a2_ref, rtb_ref, wr_ref, fcb_ref,
                out_ref):
    relc = jnp.dot(rel_ref[...].reshape(1, H), a1_ref[...],
                   preferred_element_type=jnp.float32)
    z = jnp.dot(time_ref[...], a2_ref[...],
                preferred_element_type=jnp.float32)
    z = z + relc + rtb_ref[...]
    rt = _lrelu(z)
    out = jnp.dot(rt, wr_ref[...],
                  preferred_element_type=jnp.float32) + fcb_ref[...]
    out_ref[...] = out.reshape(1, NTS, F)


def _tc_combo(rel_emb, time_emb, a1, a2, rtb, wr, fcb):
    return pl.pallas_call(
        _combo_body,
        grid=(NREL,),
        in_specs=[
            pl.BlockSpec((1, 1, H), lambda r: (r, 0, 0)),
            pl.BlockSpec((NTS, H), lambda r: (0, 0)),
            pl.BlockSpec((H, F), lambda r: (0, 0)),
            pl.BlockSpec((H, F), lambda r: (0, 0)),
            pl.BlockSpec((1, F), lambda r: (0, 0)),
            pl.BlockSpec((F, F), lambda r: (0, 0)),
            pl.BlockSpec((1, F), lambda r: (0, 0)),
        ],
        out_specs=pl.BlockSpec((1, NTS, F), lambda r: (r, 0, 0)),
        out_shape=jax.ShapeDtypeStruct((NREL, NTS, F), jnp.float32),
    )(rel_emb.reshape(NREL, 1, H), time_emb, a1, a2, rtb, wr, fcb)


_NBLK = 256
_NGRID = NP // _NBLK


def _init_body(qs_ref, qo_ref, xf_ref, ent_ref, wn_ref, nb_ref, ws_ref,
               wd_ref, x_ref, pxs_ref, pxd_ref, t0_ref, tfin_ref):
    b = pl.program_id(0)
    h = _lrelu(jnp.dot(xf_ref[...], wn_ref[...],
                       preferred_element_type=jnp.float32) + nb_ref[...])
    x = jnp.concatenate([h, ent_ref[:, pl.ds(0, H)]], axis=1)
    x_ref[...] = x
    pxs_ref[...] = jnp.dot(x, ws_ref[...], preferred_element_type=jnp.float32)
    pxd_ref[...] = jnp.dot(x, wd_ref[...], preferred_element_type=jnp.float32)
    rows = _NBLK // F
    ids = (b * _NBLK
           + lax.broadcasted_iota(jnp.int32, (rows, F), 0) * F
           + lax.broadcasted_iota(jnp.int32, (rows, F), 1))
    act = jnp.zeros((rows, F), jnp.int32)
    iso = jnp.zeros((rows, F), jnp.int32)
    for j in range(4):
        act = jnp.maximum(act, (ids == qs_ref[j]).astype(jnp.int32))
        iso = jnp.maximum(iso, (ids == qo_ref[j]).astype(jnp.int32))
    t0_ref[...] = (2 * act + (1 - iso)).reshape(1, rows, F)
    tfin_ref[...] = (2 + iso).reshape(1, rows, F)


def _tc_node_init(xf, ent_rows, wn, nb, ws, wd, q_s, q_o):
    rows = _NBLK // F
    out2 = jax.ShapeDtypeStruct((_NGRID, rows, F), jnp.int32)
    outs = [jax.ShapeDtypeStruct((NP, F), jnp.float32)] * 3 + [out2] * 2
    big = pl.BlockSpec((_NBLK, F), lambda b: (b, 0))
    tbl = pl.BlockSpec((1, rows, F), lambda b: (b, 0, 0))
    return pl.pallas_call(
        _init_body,
        grid=(_NGRID,),
        in_specs=[
            pl.BlockSpec(memory_space=pltpu.SMEM),
            pl.BlockSpec(memory_space=pltpu.SMEM),
            big,
            big,
            pl.BlockSpec((F, H), lambda b: (0, 0)),
            pl.BlockSpec((1, H), lambda b: (0, 0)),
            pl.BlockSpec((F, F), lambda b: (0, 0)),
            pl.BlockSpec((F, F), lambda b: (0, 0)),
        ],
        out_specs=[big, big, big, tbl, tbl],
        out_shape=outs,
    )(q_s, q_o, xf, ent_rows, wn, nb, ws, wd)


_CPR = NP // 8
_CPT = _CPR // NS
ECH = 64
STG = 512
NSTG = PER_W // STG


@functools.partial(
    pl.kernel,
    out_type=[
        jax.ShapeDtypeStruct((NC, NP, F), jnp.float32),
        jax.ShapeDtypeStruct((NC, NP), jnp.float32),
        jax.ShapeDtypeStruct((NC, _CPR, F), jnp.float32),
    ],
    mesh=_mesh,
    scratch_types=[
        pltpu.VMEM((STG,), jnp.int32),
        pltpu.VMEM((STG,), jnp.int32),
        pltpu.VMEM((STG,), jnp.int32),
        pltpu.VMEM((ECH,), jnp.int32),
        pltpu.VMEM((ECH,), jnp.int32),
        pltpu.VMEM((ECH,), jnp.int32),
        pltpu.VMEM((ECH,), jnp.float32),
        pltpu.VMEM((ECH, F), jnp.float32),
        pltpu.VMEM((ECH, F), jnp.float32),
        pltpu.VMEM((ECH, F), jnp.float32),
        pltpu.VMEM((RPT,), jnp.float32),
        pltpu.VMEM((NP,), jnp.int32),
        pltpu.VMEM_SHARED((NP, F), jnp.float32),
        pltpu.VMEM_SHARED((_CPR, F), jnp.float32),
        pltpu.SemaphoreType.DMA,
        pltpu.SemaphoreType.DMA,
    ],
    compiler_params=pltpu.CompilerParams(needs_layout_passes=False),
)
def _sc_edge(src_hbm, dst_hbm, cb_hbm, ctab_hbm, pxs_hbm, pxd_hbm, t_hbm,
             acc_out, cnt_out, cntp_out,
             src_v, dst_v, ci_v, si_v, si2_v, dm_v, w_v, a_v, b_v, c_v,
             ce_v, t_v, accm_sh, accc_sh, sem1, sem2):
    cid = lax.axis_index("c")
    sid = lax.axis_index("s")
    wid = sid * NC + cid
    base = wid * PER_W
    tb = sid * RPT

    pltpu.sync_copy(t_hbm, t_v)

    zeros16 = jnp.zeros((16,), jnp.float32)

    def zrow(r):
        for s in range(F // 16):
            a_v[r, pl.ds(s * 16, 16)] = zeros16

    pl.loop(0, ECH)(zrow)

    def zchunk(i):
        pltpu.sync_copy(a_v, accm_sh.at[pl.ds(tb + i * ECH, ECH)])

    pl.loop(0, RPT // ECH)(zchunk)

    def zchunk2(i):
        pltpu.sync_copy(a_v.at[pl.ds(0, 16)],
                        accc_sh.at[pl.ds(sid * _CPT + i * 16, 16)])

    pl.loop(0, _CPT // 16)(zchunk2)
    plsc.subcore_barrier()

    def stage(i):
        off = base + i * STG
        e1 = pltpu.async_copy(src_hbm.at[pl.ds(off, STG)], src_v, sem2)
        e2 = pltpu.async_copy(dst_hbm.at[pl.ds(off, STG)], dst_v, sem2)
        e3 = pltpu.async_copy(cb_hbm.at[pl.ds(off, STG)], ci_v, sem2)
        e1.wait()
        e2.wait()
        e3.wait()

        for p in range(STG // ECH):
            def wgrp(mw, g):
                sl = pl.ds(p * ECH + g * 16, 16)
                ol = pl.ds(g * 16, 16)
                sv = src_v[sl]
                dv = dst_v[sl]
                ts = plsc.load_gather(t_v, [sv])
                td = plsc.load_gather(t_v, [dv])
                ok = jnp.logical_and(ts >= 2, lax.bitwise_and(td, 1) == 1)
                w = jnp.where(ok, 1.0, 0.0)
                w_v[ol] = w
                si = jnp.where(ok, dv, TRASH)
                si_v[ol] = si
                si2_v[ol] = lax.shift_right_logical(si, 3)
                dm_v[ol] = lax.bitwise_and(dv, 7)
                return jnp.maximum(mw, jnp.max(w))

            mw = 0.0
            for g in range(ECH // 16):
                mw = wgrp(mw, g)

            @pl.when(mw > 0.0)
            def _():
                d1 = pltpu.async_copy(
                    pxs_hbm.at[src_v.at[pl.ds(p * ECH, ECH)]], a_v, sem1)
                d2 = pltpu.async_copy(
                    pxd_hbm.at[dst_v.at[pl.ds(p * ECH, ECH)]], b_v, sem1)
                d3 = pltpu.async_copy(
                    ctab_hbm.at[ci_v.at[pl.ds(p * ECH, ECH)]], c_v, sem1)
                d1.wait()
                d2.wait()
                d3.wait()

                def edge(j):
                    jb = jnp.full((16,), j, jnp.int32)
                    wb = plsc.load_gather(w_v, [jb])

                    @pl.when(jnp.max(wb) > 0.0)
                    def _():
                        for s in range(F // 16):
                            sl = pl.ds(s * 16, 16)
                            v = a_v[j, sl] + b_v[j, sl] + c_v[j, sl]
                            a_v[j, sl] = _lrelu(v) * wb
                        dmb = plsc.load_gather(dm_v, [jb])
                        for k in range(8):
                            c_v[j, pl.ds(k * 16, 16)] = jnp.where(
                                dmb == k, wb, 0.0)

                pl.loop(0, ECH)(edge)
                pltpu.sync_copy(a_v, accm_sh.at[si_v], add=True)
                pltpu.sync_copy(c_v, accc_sh.at[si2_v], add=True)

    pl.loop(0, NSTG)(stage)
    plsc.subcore_barrier()

    pltpu.sync_copy(accm_sh.at[pl.ds(tb, RPT)],
                    acc_out.at[cid, pl.ds(tb, RPT)])

    def cstage(i):
        pltpu.sync_copy(accc_sh.at[pl.ds(sid * _CPT + i * 16, 16)],
                        a_v.at[pl.ds(0, 16)])
        it = lax.iota(jnp.int32, 16)

        for g in range(8):
            ridx = lax.shift_right_logical(it + g * 16, 3)
            cidx = lax.bitwise_and(it, 7) * 16
            ce_v[pl.ds(i * CH + g * 16, 16)] = plsc.load_gather(
                a_v, [ridx, cidx])

        def crow(r):
            rb = jnp.full((16,), r, jnp.int32)
            for k in range(8):
                c_v[r, pl.ds(k * 16, 16)] = plsc.load_gather(
                    a_v, [rb, jnp.full((16,), k * 16, jnp.int32)])

        pl.loop(0, 16)(crow)
        pltpu.sync_copy(c_v.at[pl.ds(0, 16)],
                        cntp_out.at[cid, pl.ds(sid * _CPT + i * 16, 16)])

    pl.loop(0, _CPT // 16)(cstage)
    pltpu.sync_copy(ce_v, cnt_out.at[cid, pl.ds(tb, RPT)])


def _round_body(x_ref, m0_ref, m1_ref, c0_ref, c1_ref, ws_ref, wd_ref,
                xn_ref, pxs_ref, pxd_ref):
    s = m0_ref[...] + m1_ref[...]
    c = c0_ref[...] + c1_ref[...]
    cnt = jnp.max(c, axis=1, keepdims=True)
    xn = x_ref[...] + s / jnp.maximum(cnt, 1.0)
    xn_ref[...] = xn
    pxs_ref[...] = jnp.dot(xn, ws_ref[...], preferred_element_type=jnp.float32)
    pxd_ref[...] = jnp.dot(xn, wd_ref[...], preferred_element_type=jnp.float32)


def _tc_round(x, m0, m1, c0, c1, ws, wd):
    big = pl.BlockSpec((_NBLK, F), lambda b: (b, 0))
    csp = pl.BlockSpec((_NBLK, 16), lambda b: (b, 0))
    return pl.pallas_call(
        _round_body,
        grid=(_NGRID,),
        in_specs=[
            big, big, big, csp, csp,
            pl.BlockSpec((F, F), lambda b: (0, 0)),
            pl.BlockSpec((F, F), lambda b: (0, 0)),
        ],
        out_specs=[big, big, big],
        out_shape=[jax.ShapeDtypeStruct((NP, F), jnp.float32)] * 3,
    )(x, m0, m1, c0, c1, ws, wd)


def _query_body(qr_ref, qt_ref, rel_ref, time_ref, a1_ref, a2_ref, rtb_ref,
                out_ref):
    r0 = lax.broadcasted_iota(jnp.int32, (8, NREL), 0)
    ir = lax.broadcasted_iota(jnp.int32, (8, NREL), 1)
    qv = jnp.full((8, NREL), qr_ref[3], jnp.int32)
    for j in range(3):
        qv = jnp.where(r0 == j, qr_ref[j], qv)
    oh_r = (ir == qv).astype(jnp.float32)
    t0 = lax.broadcasted_iota(jnp.int32, (8, NTS), 0)
    it = lax.broadcasted_iota(jnp.int32, (8, NTS), 1)
    tv = jnp.full((8, NTS), qt_ref[3], jnp.int32)
    for j in range(3):
        tv = jnp.where(t0 == j, qt_ref[j], tv)
    oh_t = (it == tv).astype(jnp.float32)
    rel_e = jnp.dot(oh_r, rel_ref[...], preferred_element_type=jnp.float32)
    time_e = jnp.dot(oh_t, time_ref[...], preferred_element_type=jnp.float32)
    z = (jnp.dot(rel_e, a1_ref[...], preferred_element_type=jnp.float32)
         + jnp.dot(time_e, a2_ref[...], preferred_element_type=jnp.float32)
         + rtb_ref[...])
    out_ref[...] = _lrelu(z)


def _tc_query(rel_emb, time_emb, a1, a2, rtb, q_rel, q_ts):
    return pl.pallas_call(
        _query_body,
        grid=(1,),
        in_specs=[
            pl.BlockSpec(memory_space=pltpu.SMEM),
            pl.BlockSpec(memory_space=pltpu.SMEM),
            pl.BlockSpec((NREL, H), lambda b: (0, 0)),
            pl.BlockSpec((NTS, H), lambda b: (0, 0)),
            pl.BlockSpec((H, F), lambda b: (0, 0)),
            pl.BlockSpec((H, F), lambda b: (0, 0)),
            pl.BlockSpec((1, F), lambda b: (0, 0)),
        ],
        out_specs=pl.BlockSpec((8, F), lambda b: (0, 0)),
        out_shape=jax.ShapeDtypeStruct((8, F), jnp.float32),
    )(q_rel, q_ts, rel_emb, time_emb, a1, a2, rtb)


def kernel(x_feat, node_ent, edge_index, edge_type, edge_ts, src, dst,
           q_rel, q_ts, ptr, node_emb_w, node_emb_b, rel_emb_table,
           ent_emb_table, time_emb, fc_w, fc_b, rt_w, rt_b):
    i32 = jnp.int32
    f32 = jnp.float32
    q_s = (src + ptr[:-1]).astype(i32)
    q_o = (dst + ptr[:-1]).astype(i32)

    pad_e = EP - E
    srcp = jnp.concatenate([edge_index[0].astype(i32),
                            jnp.zeros((pad_e,), i32)])
    dstp = jnp.concatenate([edge_index[1].astype(i32),
                            jnp.full((pad_e,), TRASH, i32)])
    etp = jnp.concatenate([edge_type.astype(i32), jnp.zeros((pad_e,), i32)])
    etsp = jnp.concatenate([edge_ts.astype(i32), jnp.zeros((pad_e,), i32)])
    nep = jnp.concatenate([node_ent.astype(i32), jnp.zeros((NP - N,), i32)])
    xfp = jnp.concatenate([x_feat, jnp.zeros((NP - N, x_feat.shape[1]), f32)])

    entp = jnp.concatenate(
        [ent_emb_table, jnp.zeros((N, F - H), f32)], axis=1)
    combo, ent_rows = _sc_prep(etp, etsp, nep, entp)

    a1 = rt_w[:, :H].T
    a2 = rt_w[:, H:].T
    ws = fc_w[:, :F].T
    wr = fc_w[:, F:2 * F].T
    wd = fc_w[:, 2 * F:].T
    rtb2 = rt_b.reshape(1, F)
    fcb2 = fc_b.reshape(1, F)
    nb2 = node_emb_b.reshape(1, H)

    ctab = _tc_combo(rel_emb_table, time_emb, a1, a2, rtb2, wr,
                     fcb2).reshape(NCOMBO, F)

    x, pxs, pxd, t0, tfin = _tc_node_init(
        xfp, ent_rows, node_emb_w.T, nb2, ws, wd, q_s, q_o)

    t = t0.reshape(NP)
    tnot_bit = lax.bitwise_and(t, 1)
    for _ in range(3):
        accs, cnts, cntp = _sc_edge(srcp, dstp, combo, ctab, pxs, pxd, t)
        cp = cntp.reshape(NC, NP, 16)
        x, pxs, pxd = _tc_round(x, accs[0], accs[1], cp[0], cp[1], ws, wd)
        t = 2 * (cnts[0] + cnts[1] > 0.0).astype(i32) + tnot_bit

    accs, _, cntp = _sc_edge(srcp, dstp, combo, ctab, pxs, pxd,
                             tfin.reshape(NP))
    cp = cntp.reshape(NC, NP, 16)
    x, _, _ = _tc_round(x, accs[0], accs[1], cp[0], cp[1], ws, wd)

    qrt = _tc_query(rel_emb_table, time_emb, a1, a2, rtb2, q_rel.astype(i32),
                    q_ts.astype(i32))
    return (x[:N], qrt[:4])


# speedup vs baseline: 9.9866x; 1.2945x over previous
import functools

import jax
import jax.numpy as jnp
from jax import lax
from jax.experimental import pallas as pl
from jax.experimental.pallas import tpu as pltpu
from jax.experimental.pallas import tpu_sc as plsc

N = 10000
NP = 10240
E = 320000
H = 64
F = 2 * H
NREL = 200
NTS = 365
NCOMBO = NREL * NTS

NC = 2
NS = 16
NWORK = NC * NS
CH = 128
NCHUNK = 80
PER_W = CH * NCHUNK
EP = PER_W * NWORK
TRASH = NP - 1
RPT = NP // NS
ACCW = F + 16

_mesh = plsc.VectorSubcoreMesh(
    core_axis_name="c", subcore_axis_name="s", num_cores=NC, num_subcores=NS)


def _lrelu(v):
    return jnp.maximum(v, 0.2 * v)


@functools.partial(
    pl.kernel,
    out_type=[
        jax.ShapeDtypeStruct((EP,), jnp.int32),
        jax.ShapeDtypeStruct((NP, F), jnp.float32),
    ],
    mesh=_mesh,
    scratch_types=[
        pltpu.VMEM((CH,), jnp.int32),
        pltpu.VMEM((CH,), jnp.int32),
        pltpu.VMEM((CH,), jnp.int32),
        pltpu.VMEM((80,), jnp.int32),
        pltpu.VMEM((80, F), jnp.float32),
        pltpu.SemaphoreType.DMA,
    ],
    compiler_params=pltpu.CompilerParams(needs_layout_passes=False),
)
def _sc_prep(et_hbm, ets_hbm, ne_hbm, ent_tab_hbm, combo_out, ent_out,
             et_v, ets_v, cb_v, ni_v, er_v, sem):
    cid = lax.axis_index("c")
    sid = lax.axis_index("s")
    wid = sid * NC + cid
    base = wid * PER_W

    def chunk(i):
        off = base + i * CH
        pltpu.sync_copy(et_hbm.at[pl.ds(off, CH)], et_v)
        pltpu.sync_copy(ets_hbm.at[pl.ds(off, CH)], ets_v)
        for g in range(CH // 16):
            sl = pl.ds(g * 16, 16)
            cb_v[sl] = et_v[sl] * NTS + ets_v[sl]
        pltpu.sync_copy(cb_v, combo_out.at[pl.ds(off, CH)])

    pl.loop(0, NCHUNK)(chunk)

    def nchunk(i):
        off = wid * (NP // NWORK) + i * 80
        pltpu.sync_copy(ne_hbm.at[pl.ds(off, 80)], ni_v)
        pltpu.async_copy(ent_tab_hbm.at[ni_v], er_v, sem).wait()
        pltpu.sync_copy(er_v, ent_out.at[pl.ds(off, 80)])

    pl.loop(0, (NP // NWORK) // 80)(nchunk)


def _combo_body(rel_ref, time_ref, a1_ref, a2_ref, rtb_ref, wr_ref, fcb_ref,
                out_ref):
    relc = jnp.dot(rel_ref[...].reshape(1, H), a1_ref[...],
                   preferred_element_type=jnp.float32)
    z = jnp.dot(time_ref[...], a2_ref[...],
                preferred_element_type=jnp.float32)
    z = z + relc + rtb_ref[...]
    rt = _lrelu(z)
    out = jnp.dot(rt, wr_ref[...],
                  preferred_element_type=jnp.float32) + fcb_ref[...]
    out_ref[...] = out.reshape(1, NTS, F)


def _tc_combo(rel_emb, time_emb, a1, a2, rtb, wr, fcb):
    return pl.pallas_call(
        _combo_body,
        grid=(NREL,),
        in_specs=[
            pl.BlockSpec((1, 1, H), lambda r: (r, 0, 0)),
            pl.BlockSpec((NTS, H), lambda r: (0, 0)),
            pl.BlockSpec((H, F), lambda r: (0, 0)),
            pl.BlockSpec((H, F), lambda r: (0, 0)),
            pl.BlockSpec((1, F), lambda r: (0, 0)),
            pl.BlockSpec((F, F), lambda r: (0, 0)),
            pl.BlockSpec((1, F), lambda r: (0, 0)),
        ],
        out_specs=pl.BlockSpec((1, NTS, F), lambda r: (r, 0, 0)),
        out_shape=jax.ShapeDtypeStruct((NREL, NTS, F), jnp.float32),
    )(rel_emb.reshape(NREL, 1, H), time_emb, a1, a2, rtb, wr, fcb)


_NBLK = 256
_NGRID = NP // _NBLK


def _init_body(qs_ref, qo_ref, xf_ref, ent_ref, wn_ref, nb_ref, ws_ref,
               wd_ref, x_ref, pxs_ref, pxd_ref, t0_ref, tfin_ref):
    b = pl.program_id(0)
    h = _lrelu(jnp.dot(xf_ref[...], wn_ref[...],
                       preferred_element_type=jnp.float32) + nb_ref[...])
    x = jnp.concatenate([h, ent_ref[:, pl.ds(0, H)]], axis=1)
    x_ref[...] = x
    pxs_ref[...] = jnp.dot(x, ws_ref[...], preferred_element_type=jnp.float32)
    pxd_ref[...] = jnp.dot(x, wd_ref[...], preferred_element_type=jnp.float32)
    rows = _NBLK // F
    ids = (b * _NBLK
           + lax.broadcasted_iota(jnp.int32, (rows, F), 0) * F
           + lax.broadcasted_iota(jnp.int32, (rows, F), 1))
    act = jnp.zeros((rows, F), jnp.int32)
    iso = jnp.zeros((rows, F), jnp.int32)
    for j in range(4):
        act = jnp.maximum(act, (ids == qs_ref[j]).astype(jnp.int32))
        iso = jnp.maximum(iso, (ids == qo_ref[j]).astype(jnp.int32))
    t0_ref[...] = (2 * act + (1 - iso)).reshape(1, rows, F)
    tfin_ref[...] = (2 + iso).reshape(1, rows, F)


def _tc_node_init(xf, ent_rows, wn, nb, ws, wd, q_s, q_o):
    rows = _NBLK // F
    out2 = jax.ShapeDtypeStruct((_NGRID, rows, F), jnp.int32)
    outs = [jax.ShapeDtypeStruct((NP, F), jnp.float32)] * 3 + [out2] * 2
    big = pl.BlockSpec((_NBLK, F), lambda b: (b, 0))
    tbl = pl.BlockSpec((1, rows, F), lambda b: (b, 0, 0))
    return pl.pallas_call(
        _init_body,
        grid=(_NGRID,),
        in_specs=[
            pl.BlockSpec(memory_space=pltpu.SMEM),
            pl.BlockSpec(memory_space=pltpu.SMEM),
            big,
            big,
            pl.BlockSpec((F, H), lambda b: (0, 0)),
            pl.BlockSpec((1, H), lambda b: (0, 0)),
            pl.BlockSpec((F, F), lambda b: (0, 0)),
            pl.BlockSpec((F, F), lambda b: (0, 0)),
        ],
        out_specs=[big, big, big, tbl, tbl],
        out_shape=outs,
    )(q_s, q_o, xf, ent_rows, wn, nb, ws, wd)


_CPR = NP // 8
_CPT = _CPR // NS
ECH = 32
STG = 512
NSTG = PER_W // STG


@functools.partial(
    pl.kernel,
    out_type=[
        jax.ShapeDtypeStruct((NC, NP, F), jnp.float32),
        jax.ShapeDtypeStruct((NC, NP), jnp.float32),
        jax.ShapeDtypeStruct((NC, _CPR, F), jnp.float32),
    ],
    mesh=_mesh,
    scratch_types=[
        pltpu.VMEM((STG,), jnp.int32),
        pltpu.VMEM((STG,), jnp.int32),
        pltpu.VMEM((STG,), jnp.int32),
        pltpu.VMEM((2, ECH), jnp.int32),
        pltpu.VMEM((2, ECH), jnp.int32),
        pltpu.VMEM((2 * ECH,), jnp.int32),
        pltpu.VMEM((2 * ECH,), jnp.float32),
        pltpu.VMEM((2, ECH, F), jnp.float32),
        pltpu.VMEM((2, ECH, F), jnp.float32),
        pltpu.VMEM((2, ECH, F), jnp.float32),
        pltpu.VMEM((RPT,), jnp.float32),
        pltpu.VMEM((NP,), jnp.int32),
        pltpu.VMEM_SHARED((NP, F), jnp.float32),
        pltpu.VMEM_SHARED((_CPR, F), jnp.float32),
        pltpu.SemaphoreType.DMA,
        pltpu.SemaphoreType.DMA,
    ],
    compiler_params=pltpu.CompilerParams(needs_layout_passes=False),
)
def _sc_edge(src_hbm, dst_hbm, cb_hbm, ctab_hbm, pxs_hbm, pxd_hbm, t_hbm,
             acc_out, cnt_out, cntp_out,
             src_v, dst_v, ci_v, si_v, si2_v, dm_v, w_v, a_v, b_v, c_v,
             ce_v, t_v, accm_sh, accc_sh, sem1, sem2):
    cid = lax.axis_index("c")
    sid = lax.axis_index("s")
    wid = sid * NC + cid
    base = wid * PER_W
    tb = sid * RPT

    pltpu.sync_copy(t_hbm, t_v)

    zeros16 = jnp.zeros((16,), jnp.float32)

    def zrow(r):
        for s in range(F // 16):
            a_v[0, r, pl.ds(s * 16, 16)] = zeros16

    pl.loop(0, ECH)(zrow)

    def zchunk(i):
        pltpu.sync_copy(a_v.at[0], accm_sh.at[pl.ds(tb + i * ECH, ECH)])

    pl.loop(0, RPT // ECH)(zchunk)

    def zchunk2(i):
        pltpu.sync_copy(a_v.at[0].at[pl.ds(0, 16)],
                        accc_sh.at[pl.ds(sid * _CPT + i * 16, 16)])

    pl.loop(0, _CPT // 16)(zchunk2)
    plsc.subcore_barrier()

    nchk = STG // ECH

    def stage(i):
        off = base + i * STG
        e1 = pltpu.async_copy(src_hbm.at[pl.ds(off, STG)], src_v, sem2)
        e2 = pltpu.async_copy(dst_hbm.at[pl.ds(off, STG)], dst_v, sem2)
        e3 = pltpu.async_copy(cb_hbm.at[pl.ds(off, STG)], ci_v, sem2)
        e1.wait()
        e2.wait()
        e3.wait()

        def wpass(p):
            buf = p % 2
            mw = 0.0
            for g in range(ECH // 16):
                sl = pl.ds(p * ECH + g * 16, 16)
                gl = pl.ds(g * 16, 16)
                ol = pl.ds(buf * ECH + g * 16, 16)
                sv = src_v[sl]
                dv = dst_v[sl]
                ts = plsc.load_gather(t_v, [sv])
                td = plsc.load_gather(t_v, [dv])
                ok = jnp.logical_and(ts >= 2, lax.bitwise_and(td, 1) == 1)
                w = jnp.where(ok, 1.0, 0.0)
                w_v[ol] = w
                si = jnp.where(ok, dv, TRASH)
                si_v[buf, gl] = si
                si2_v[buf, gl] = lax.shift_right_logical(si, 3)
                dm_v[ol] = lax.bitwise_and(dv, 7)
                mw = jnp.maximum(mw, jnp.max(w))
            return mw

        def gxfer(p, issue):
            buf = p % 2
            ds = pl.ds(p * ECH, ECH)
            trips = [
                (pxs_hbm.at[src_v.at[ds]], a_v.at[buf]),
                (pxd_hbm.at[dst_v.at[ds]], b_v.at[buf]),
                (ctab_hbm.at[ci_v.at[ds]], c_v.at[buf]),
            ]
            for s_ref, d_ref in trips:
                if issue:
                    pltpu.async_copy(s_ref, d_ref, sem1)
                else:
                    pltpu.make_async_copy(s_ref, d_ref, sem1).wait()

        def compute(q):
            buf = q % 2

            def edge(j):
                jb = jnp.full((16,), buf * ECH + j, jnp.int32)
                wb = plsc.load_gather(w_v, [jb])

                @pl.when(jnp.max(wb) > 0.0)
                def _():
                    for s in range(F // 16):
                        sl = pl.ds(s * 16, 16)
                        v = a_v[buf, j, sl] + b_v[buf, j, sl] + c_v[buf, j, sl]
                        a_v[buf, j, sl] = _lrelu(v) * wb
                    dmb = plsc.load_gather(dm_v, [jb])
                    for k in range(8):
                        c_v[buf, j, pl.ds(k * 16, 16)] = jnp.where(
                            dmb == k, wb, 0.0)

            pl.loop(0, ECH)(edge)
            pltpu.sync_copy(a_v.at[buf], accm_sh.at[si_v.at[buf]], add=True)
            pltpu.sync_copy(c_v.at[buf], accc_sh.at[si2_v.at[buf]], add=True)

        mws = []
        for p in range(nchk + 1):
            if p < nchk:
                mw = wpass(p)
                mws.append(mw)
                pl.when(mw > 0.0)(functools.partial(gxfer, p, True))
            if p >= 1:
                q = p - 1

                def fin(q=q):
                    gxfer(q, False)
                    compute(q)

                pl.when(mws[q] > 0.0)(fin)

    pl.loop(0, NSTG)(stage)
    plsc.subcore_barrier()

    pltpu.sync_copy(accm_sh.at[pl.ds(tb, RPT)],
                    acc_out.at[cid, pl.ds(tb, RPT)])

    def cstage(i):
        pltpu.sync_copy(accc_sh.at[pl.ds(sid * _CPT + i * 16, 16)],
                        a_v.at[0].at[pl.ds(0, 16)])
        it = lax.iota(jnp.int32, 16)

        for g in range(8):
            ridx = lax.shift_right_logical(it + g * 16, 3)
            cidx = lax.bitwise_and(it, 7) * 16
            ce_v[pl.ds(i * CH + g * 16, 16)] = plsc.load_gather(
                a_v.at[0], [ridx, cidx])

        def crow(r):
            rb = jnp.full((16,), r, jnp.int32)
            for k in range(8):
                c_v[0, r, pl.ds(k * 16, 16)] = plsc.load_gather(
                    a_v.at[0], [rb, jnp.full((16,), k * 16, jnp.int32)])

        pl.loop(0, 16)(crow)
        pltpu.sync_copy(c_v.at[0].at[pl.ds(0, 16)],
                        cntp_out.at[cid, pl.ds(sid * _CPT + i * 16, 16)])

    pl.loop(0, _CPT // 16)(cstage)
    pltpu.sync_copy(ce_v, cnt_out.at[cid, pl.ds(tb, RPT)])


def _round_body(x_ref, m0_ref, m1_ref, c0_ref, c1_ref, ws_ref, wd_ref,
                xn_ref, pxs_ref, pxd_ref):
    s = m0_ref[...] + m1_ref[...]
    c = c0_ref[...] + c1_ref[...]
    cnt = jnp.max(c, axis=1, keepdims=True)
    xn = x_ref[...] + s / jnp.maximum(cnt, 1.0)
    xn_ref[...] = xn
    pxs_ref[...] = jnp.dot(xn, ws_ref[...], preferred_element_type=jnp.float32)
    pxd_ref[...] = jnp.dot(xn, wd_ref[...], preferred_element_type=jnp.float32)


def _tc_round(x, m0, m1, c0, c1, ws, wd):
    big = pl.BlockSpec((_NBLK, F), lambda b: (b, 0))
    csp = pl.BlockSpec((_NBLK, 16), lambda b: (b, 0))
    return pl.pallas_call(
        _round_body,
        grid=(_NGRID,),
        in_specs=[
            big, big, big, csp, csp,
            pl.BlockSpec((F, F), lambda b: (0, 0)),
            pl.BlockSpec((F, F), lambda b: (0, 0)),
        ],
        out_specs=[big, big, big],
        out_shape=[jax.ShapeDtypeStruct((NP, F), jnp.float32)] * 3,
    )(x, m0, m1, c0, c1, ws, wd)


def _query_body(qr_ref, qt_ref, rel_ref, time_ref, a1_ref, a2_ref, rtb_ref,
                out_ref):
    r0 = lax.broadcasted_iota(jnp.int32, (8, NREL), 0)
    ir = lax.broadcasted_iota(jnp.int32, (8, NREL), 1)
    qv = jnp.full((8, NREL), qr_ref[3], jnp.int32)
    for j in range(3):
        qv = jnp.where(r0 == j, qr_ref[j], qv)
    oh_r = (ir == qv).astype(jnp.float32)
    t0 = lax.broadcasted_iota(jnp.int32, (8, NTS), 0)
    it = lax.broadcasted_iota(jnp.int32, (8, NTS), 1)
    tv = jnp.full((8, NTS), qt_ref[3], jnp.int32)
    for j in range(3):
        tv = jnp.where(t0 == j, qt_ref[j], tv)
    oh_t = (it == tv).astype(jnp.float32)
    rel_e = jnp.dot(oh_r, rel_ref[...], preferred_element_type=jnp.float32)
    time_e = jnp.dot(oh_t, time_ref[...], preferred_element_type=jnp.float32)
    z = (jnp.dot(rel_e, a1_ref[...], preferred_element_type=jnp.float32)
         + jnp.dot(time_e, a2_ref[...], preferred_element_type=jnp.float32)
         + rtb_ref[...])
    out_ref[...] = _lrelu(z)


def _tc_query(rel_emb, time_emb, a1, a2, rtb, q_rel, q_ts):
    return pl.pallas_call(
        _query_body,
        grid=(1,),
        in_specs=[
            pl.BlockSpec(memory_space=pltpu.SMEM),
            pl.BlockSpec(memory_space=pltpu.SMEM),
            pl.BlockSpec((NREL, H), lambda b: (0, 0)),
            pl.BlockSpec((NTS, H), lambda b: (0, 0)),
            pl.BlockSpec((H, F), lambda b: (0, 0)),
            pl.BlockSpec((H, F), lambda b: (0, 0)),
            pl.BlockSpec((1, F), lambda b: (0, 0)),
        ],
        out_specs=pl.BlockSpec((8, F), lambda b: (0, 0)),
        out_shape=jax.ShapeDtypeStruct((8, F), jnp.float32),
    )(q_rel, q_ts, rel_emb, time_emb, a1, a2, rtb)


def kernel(x_feat, node_ent, edge_index, edge_type, edge_ts, src, dst,
           q_rel, q_ts, ptr, node_emb_w, node_emb_b, rel_emb_table,
           ent_emb_table, time_emb, fc_w, fc_b, rt_w, rt_b):
    i32 = jnp.int32
    f32 = jnp.float32
    q_s = (src + ptr[:-1]).astype(i32)
    q_o = (dst + ptr[:-1]).astype(i32)

    pad_e = EP - E
    srcp = jnp.concatenate([edge_index[0].astype(i32),
                            jnp.zeros((pad_e,), i32)])
    dstp = jnp.concatenate([edge_index[1].astype(i32),
                            jnp.full((pad_e,), TRASH, i32)])
    etp = jnp.concatenate([edge_type.astype(i32), jnp.zeros((pad_e,), i32)])
    etsp = jnp.concatenate([edge_ts.astype(i32), jnp.zeros((pad_e,), i32)])
    nep = jnp.concatenate([node_ent.astype(i32), jnp.zeros((NP - N,), i32)])
    xfp = jnp.concatenate([x_feat, jnp.zeros((NP - N, x_feat.shape[1]), f32)])

    entp = jnp.concatenate(
        [ent_emb_table, jnp.zeros((N, F - H), f32)], axis=1)
    combo, ent_rows = _sc_prep(etp, etsp, nep, entp)

    a1 = rt_w[:, :H].T
    a2 = rt_w[:, H:].T
    ws = fc_w[:, :F].T
    wr = fc_w[:, F:2 * F].T
    wd = fc_w[:, 2 * F:].T
    rtb2 = rt_b.reshape(1, F)
    fcb2 = fc_b.reshape(1, F)
    nb2 = node_emb_b.reshape(1, H)

    ctab = _tc_combo(rel_emb_table, time_emb, a1, a2, rtb2, wr,
                     fcb2).reshape(NCOMBO, F)

    x, pxs, pxd, t0, tfin = _tc_node_init(
        xfp, ent_rows, node_emb_w.T, nb2, ws, wd, q_s, q_o)

    t = t0.reshape(NP)
    tnot_bit = lax.bitwise_and(t, 1)
    for _ in range(3):
        accs, cnts, cntp = _sc_edge(srcp, dstp, combo, ctab, pxs, pxd, t)
        cp = cntp.reshape(NC, NP, 16)
        x, pxs, pxd = _tc_round(x, accs[0], accs[1], cp[0], cp[1], ws, wd)
        t = 2 * (cnts[0] + cnts[1] > 0.0).astype(i32) + tnot_bit

    accs, _, cntp = _sc_edge(srcp, dstp, combo, ctab, pxs, pxd,
                             tfin.reshape(NP))
    cp = cntp.reshape(NC, NP, 16)
    x, _, _ = _tc_round(x, accs[0], accs[1], cp[0], cp[1], ws, wd)

    qrt = _tc_query(rel_emb_table, time_emb, a1, a2, rtb2, q_rel.astype(i32),
                    q_ts.astype(i32))
    return (x[:N], qrt[:4])


# speedup vs baseline: 10.1804x; 1.0194x over previous
import functools

import jax
import jax.numpy as jnp
from jax import lax
from jax.experimental import pallas as pl
from jax.experimental.pallas import tpu as pltpu
from jax.experimental.pallas import tpu_sc as plsc

N = 10000
NP = 10240
E = 320000
H = 64
F = 2 * H
NREL = 200
NTS = 365
NCOMBO = NREL * NTS

NC = 2
NS = 16
NWORK = NC * NS
CH = 128
NCHUNK = 80
PER_W = CH * NCHUNK
EP = PER_W * NWORK
TRASH = NP - 1
RPT = NP // NS
ACCW = F + 16

_mesh = plsc.VectorSubcoreMesh(
    core_axis_name="c", subcore_axis_name="s", num_cores=NC, num_subcores=NS)


def _lrelu(v):
    return jnp.maximum(v, 0.2 * v)


@functools.partial(
    pl.kernel,
    out_type=[
        jax.ShapeDtypeStruct((EP,), jnp.int32),
        jax.ShapeDtypeStruct((NP, F), jnp.float32),
    ],
    mesh=_mesh,
    scratch_types=[
        pltpu.VMEM((CH,), jnp.int32),
        pltpu.VMEM((CH,), jnp.int32),
        pltpu.VMEM((CH,), jnp.int32),
        pltpu.VMEM((80,), jnp.int32),
        pltpu.VMEM((80, F), jnp.float32),
        pltpu.SemaphoreType.DMA,
    ],
    compiler_params=pltpu.CompilerParams(needs_layout_passes=False),
)
def _sc_prep(et_hbm, ets_hbm, ne_hbm, ent_tab_hbm, combo_out, ent_out,
             et_v, ets_v, cb_v, ni_v, er_v, sem):
    cid = lax.axis_index("c")
    sid = lax.axis_index("s")
    wid = sid * NC + cid
    base = wid * PER_W

    def chunk(i):
        off = base + i * CH
        pltpu.sync_copy(et_hbm.at[pl.ds(off, CH)], et_v)
        pltpu.sync_copy(ets_hbm.at[pl.ds(off, CH)], ets_v)
        for g in range(CH // 16):
            sl = pl.ds(g * 16, 16)
            cb_v[sl] = et_v[sl] * NTS + ets_v[sl]
        pltpu.sync_copy(cb_v, combo_out.at[pl.ds(off, CH)])

    pl.loop(0, NCHUNK)(chunk)

    def nchunk(i):
        off = wid * (NP // NWORK) + i * 80
        pltpu.sync_copy(ne_hbm.at[pl.ds(off, 80)], ni_v)
        pltpu.async_copy(ent_tab_hbm.at[ni_v], er_v, sem).wait()
        pltpu.sync_copy(er_v, ent_out.at[pl.ds(off, 80)])

    pl.loop(0, (NP // NWORK) // 80)(nchunk)


def _combo_body(rel_ref, time_ref, a1_ref, a2_ref, rtb_ref, wr_ref, fcb_ref,
                out_ref):
    relc = jnp.dot(rel_ref[...].reshape(1, H), a1_ref[...],
                   preferred_element_type=jnp.float32)
    z = jnp.dot(time_ref[...], a2_ref[...],
                preferred_element_type=jnp.float32)
    z = z + relc + rtb_ref[...]
    rt = _lrelu(z)
    out = jnp.dot(rt, wr_ref[...],
                  preferred_element_type=jnp.float32) + fcb_ref[...]
    out_ref[...] = out.reshape(1, NTS, F)


def _tc_combo(rel_emb, time_emb, a1, a2, rtb, wr, fcb):
    return pl.pallas_call(
        _combo_body,
        grid=(NREL,),
        in_specs=[
            pl.BlockSpec((1, 1, H), lambda r: (r, 0, 0)),
            pl.BlockSpec((NTS, H), lambda r: (0, 0)),
            pl.BlockSpec((H, F), lambda r: (0, 0)),
            pl.BlockSpec((H, F), lambda r: (0, 0)),
            pl.BlockSpec((1, F), lambda r: (0, 0)),
            pl.BlockSpec((F, F), lambda r: (0, 0)),
            pl.BlockSpec((1, F), lambda r: (0, 0)),
        ],
        out_specs=pl.BlockSpec((1, NTS, F), lambda r: (r, 0, 0)),
        out_shape=jax.ShapeDtypeStruct((NREL, NTS, F), jnp.float32),
    )(rel_emb.reshape(NREL, 1, H), time_emb, a1, a2, rtb, wr, fcb)


_NBLK = 256
_NGRID = NP // _NBLK


def _init_body(qs_ref, qo_ref, xf_ref, ent_ref, wn_ref, nb_ref, ws_ref,
               wd_ref, x_ref, pxs_ref, pxd_ref, t0_ref, tfin_ref):
    b = pl.program_id(0)
    h = _lrelu(jnp.dot(xf_ref[...], wn_ref[...],
                       preferred_element_type=jnp.float32) + nb_ref[...])
    x = jnp.concatenate([h, ent_ref[:, pl.ds(0, H)]], axis=1)
    x_ref[...] = x
    pxs_ref[...] = jnp.dot(x, ws_ref[...], preferred_element_type=jnp.float32)
    pxd_ref[...] = jnp.dot(x, wd_ref[...], preferred_element_type=jnp.float32)
    rows = _NBLK // F
    ids = (b * _NBLK
           + lax.broadcasted_iota(jnp.int32, (rows, F), 0) * F
           + lax.broadcasted_iota(jnp.int32, (rows, F), 1))
    act = jnp.zeros((rows, F), jnp.int32)
    iso = jnp.zeros((rows, F), jnp.int32)
    for j in range(4):
        act = jnp.maximum(act, (ids == qs_ref[j]).astype(jnp.int32))
        iso = jnp.maximum(iso, (ids == qo_ref[j]).astype(jnp.int32))
    t0_ref[...] = (2 * act + (1 - iso)).reshape(1, rows, F)
    tfin_ref[...] = (2 + iso).reshape(1, rows, F)


def _tc_node_init(xf, ent_rows, wn, nb, ws, wd, q_s, q_o):
    rows = _NBLK // F
    out2 = jax.ShapeDtypeStruct((_NGRID, rows, F), jnp.int32)
    outs = [jax.ShapeDtypeStruct((NP, F), jnp.float32)] * 3 + [out2] * 2
    big = pl.BlockSpec((_NBLK, F), lambda b: (b, 0))
    tbl = pl.BlockSpec((1, rows, F), lambda b: (b, 0, 0))
    return pl.pallas_call(
        _init_body,
        grid=(_NGRID,),
        in_specs=[
            pl.BlockSpec(memory_space=pltpu.SMEM),
            pl.BlockSpec(memory_space=pltpu.SMEM),
            big,
            big,
            pl.BlockSpec((F, H), lambda b: (0, 0)),
            pl.BlockSpec((1, H), lambda b: (0, 0)),
            pl.BlockSpec((F, F), lambda b: (0, 0)),
            pl.BlockSpec((F, F), lambda b: (0, 0)),
        ],
        out_specs=[big, big, big, tbl, tbl],
        out_shape=outs,
    )(q_s, q_o, xf, ent_rows, wn, nb, ws, wd)


_CPR = NP // 8
_CPT = _CPR // NS
ECH = 32
STG = 512
NSTG = PER_W // STG


@functools.partial(
    pl.kernel,
    out_type=[
        jax.ShapeDtypeStruct((NC, NP, F), jnp.float32),
        jax.ShapeDtypeStruct((NC, NP), jnp.float32),
        jax.ShapeDtypeStruct((NC, _CPR, F), jnp.float32),
    ],
    mesh=_mesh,
    scratch_types=[
        pltpu.VMEM((STG,), jnp.int32),
        pltpu.VMEM((STG,), jnp.int32),
        pltpu.VMEM((STG,), jnp.int32),
        pltpu.VMEM((2, ECH), jnp.int32),
        pltpu.VMEM((2, ECH), jnp.int32),
        pltpu.VMEM((2 * ECH,), jnp.int32),
        pltpu.VMEM((2 * ECH,), jnp.float32),
        pltpu.VMEM((2, ECH, F), jnp.float32),
        pltpu.VMEM((2, ECH, F), jnp.float32),
        pltpu.VMEM((2, ECH, F), jnp.float32),
        pltpu.VMEM((RPT,), jnp.float32),
        pltpu.VMEM((NP,), jnp.int32),
        pltpu.VMEM_SHARED((NP, F), jnp.float32),
        pltpu.VMEM_SHARED((_CPR, F), jnp.float32),
        pltpu.SemaphoreType.DMA,
        pltpu.SemaphoreType.DMA,
        pltpu.SemaphoreType.DMA,
    ],
    compiler_params=pltpu.CompilerParams(needs_layout_passes=False),
)
def _sc_edge(src_hbm, dst_hbm, cb_hbm, ctab_hbm, pxs_hbm, pxd_hbm, t_hbm,
             acc_out, cnt_out, cntp_out,
             src_v, dst_v, ci_v, si_v, si2_v, dm_v, w_v, a_v, b_v, c_v,
             ce_v, t_v, accm_sh, accc_sh, sem1, sem2, sem3):
    cid = lax.axis_index("c")
    sid = lax.axis_index("s")
    wid = sid * NC + cid
    base = wid * PER_W
    tb = sid * RPT

    pltpu.sync_copy(t_hbm, t_v)

    zeros16 = jnp.zeros((16,), jnp.float32)

    def zrow(r):
        for s in range(F // 16):
            a_v[0, r, pl.ds(s * 16, 16)] = zeros16

    pl.loop(0, ECH)(zrow)

    def zchunk(i):
        pltpu.sync_copy(a_v.at[0], accm_sh.at[pl.ds(tb + i * ECH, ECH)])

    pl.loop(0, RPT // ECH)(zchunk)

    def zchunk2(i):
        pltpu.sync_copy(a_v.at[0].at[pl.ds(0, 16)],
                        accc_sh.at[pl.ds(sid * _CPT + i * 16, 16)])

    pl.loop(0, _CPT // 16)(zchunk2)
    plsc.subcore_barrier()

    nchk = STG // ECH

    def stage(i):
        off = base + i * STG
        e1 = pltpu.async_copy(src_hbm.at[pl.ds(off, STG)], src_v, sem2)
        e2 = pltpu.async_copy(dst_hbm.at[pl.ds(off, STG)], dst_v, sem2)
        e3 = pltpu.async_copy(cb_hbm.at[pl.ds(off, STG)], ci_v, sem2)
        e1.wait()
        e2.wait()
        e3.wait()

        def wpass(p):
            buf = p % 2
            mw = 0.0
            for g in range(ECH // 16):
                sl = pl.ds(p * ECH + g * 16, 16)
                gl = pl.ds(g * 16, 16)
                ol = pl.ds(buf * ECH + g * 16, 16)
                sv = src_v[sl]
                dv = dst_v[sl]
                ts = plsc.load_gather(t_v, [sv])
                td = plsc.load_gather(t_v, [dv])
                ok = jnp.logical_and(ts >= 2, lax.bitwise_and(td, 1) == 1)
                w = jnp.where(ok, 1.0, 0.0)
                w_v[ol] = w
                si = jnp.where(ok, dv, TRASH)
                si_v[buf, gl] = si
                si2_v[buf, gl] = lax.shift_right_logical(si, 3)
                dm_v[ol] = lax.bitwise_and(dv, 7)
                mw = jnp.maximum(mw, jnp.max(w))
            return mw

        def gxfer(p, issue):
            buf = p % 2
            ds = pl.ds(p * ECH, ECH)
            trips = [
                (pxs_hbm.at[src_v.at[ds]], a_v.at[buf]),
                (pxd_hbm.at[dst_v.at[ds]], b_v.at[buf]),
                (ctab_hbm.at[ci_v.at[ds]], c_v.at[buf]),
            ]
            for s_ref, d_ref in trips:
                if issue:
                    pltpu.async_copy(s_ref, d_ref, sem1)
                else:
                    pltpu.make_async_copy(s_ref, d_ref, sem1).wait()

        def compute(q):
            buf = q % 2

            def edge(j):
                jb = jnp.full((16,), buf * ECH + j, jnp.int32)
                wb = plsc.load_gather(w_v, [jb])

                @pl.when(jnp.max(wb) > 0.0)
                def _():
                    for s in range(F // 16):
                        sl = pl.ds(s * 16, 16)
                        v = a_v[buf, j, sl] + b_v[buf, j, sl] + c_v[buf, j, sl]
                        a_v[buf, j, sl] = _lrelu(v) * wb
                    dmb = plsc.load_gather(dm_v, [jb])
                    for k in range(8):
                        c_v[buf, j, pl.ds(k * 16, 16)] = jnp.where(
                            dmb == k, wb, 0.0)

            pl.loop(0, ECH)(edge)
            pltpu.async_copy(a_v.at[buf], accm_sh.at[si_v.at[buf]], sem3,
                             add=True)
            pltpu.async_copy(c_v.at[buf], accc_sh.at[si2_v.at[buf]], sem3,
                             add=True)

        def sdrain(q):
            buf = q % 2
            pltpu.make_async_copy(a_v.at[buf], accm_sh.at[si_v.at[buf]],
                                  sem3).wait()
            pltpu.make_async_copy(c_v.at[buf], accc_sh.at[si2_v.at[buf]],
                                  sem3).wait()

        mws = []
        for p in range(nchk + 1):
            if p < nchk:
                if p >= 2:
                    pl.when(mws[p - 2] > 0.0)(functools.partial(sdrain, p - 2))
                mw = wpass(p)
                mws.append(mw)
                pl.when(mw > 0.0)(functools.partial(gxfer, p, True))
            if p >= 1:
                q = p - 1

                def fin(q=q):
                    gxfer(q, False)
                    compute(q)

                pl.when(mws[q] > 0.0)(fin)
        for q in (nchk - 2, nchk - 1):
            pl.when(mws[q] > 0.0)(functools.partial(sdrain, q))

    pl.loop(0, NSTG)(stage)
    plsc.subcore_barrier()

    pltpu.sync_copy(accm_sh.at[pl.ds(tb, RPT)],
                    acc_out.at[cid, pl.ds(tb, RPT)])

    def cstage(i):
        pltpu.sync_copy(accc_sh.at[pl.ds(sid * _CPT + i * 16, 16)],
                        a_v.at[0].at[pl.ds(0, 16)])
        it = lax.iota(jnp.int32, 16)

        for g in range(8):
            ridx = lax.shift_right_logical(it + g * 16, 3)
            cidx = lax.bitwise_and(it, 7) * 16
            ce_v[pl.ds(i * CH + g * 16, 16)] = plsc.load_gather(
                a_v.at[0], [ridx, cidx])

        def crow(r):
            rb = jnp.full((16,), r, jnp.int32)
            for k in range(8):
                c_v[0, r, pl.ds(k * 16, 16)] = plsc.load_gather(
                    a_v.at[0], [rb, jnp.full((16,), k * 16, jnp.int32)])

        pl.loop(0, 16)(crow)
        pltpu.sync_copy(c_v.at[0].at[pl.ds(0, 16)],
                        cntp_out.at[cid, pl.ds(sid * _CPT + i * 16, 16)])

    pl.loop(0, _CPT // 16)(cstage)
    pltpu.sync_copy(ce_v, cnt_out.at[cid, pl.ds(tb, RPT)])


def _round_body(x_ref, m0_ref, m1_ref, c0_ref, c1_ref, ws_ref, wd_ref,
                xn_ref, pxs_ref, pxd_ref):
    s = m0_ref[...] + m1_ref[...]
    c = c0_ref[...] + c1_ref[...]
    cnt = jnp.max(c, axis=1, keepdims=True)
    xn = x_ref[...] + s / jnp.maximum(cnt, 1.0)
    xn_ref[...] = xn
    pxs_ref[...] = jnp.dot(xn, ws_ref[...], preferred_element_type=jnp.float32)
    pxd_ref[...] = jnp.dot(xn, wd_ref[...], preferred_element_type=jnp.float32)


def _tc_round(x, m0, m1, c0, c1, ws, wd):
    big = pl.BlockSpec((_NBLK, F), lambda b: (b, 0))
    csp = pl.BlockSpec((_NBLK, 16), lambda b: (b, 0))
    return pl.pallas_call(
        _round_body,
        grid=(_NGRID,),
        in_specs=[
            big, big, big, csp, csp,
            pl.BlockSpec((F, F), lambda b: (0, 0)),
            pl.BlockSpec((F, F), lambda b: (0, 0)),
        ],
        out_specs=[big, big, big],
        out_shape=[jax.ShapeDtypeStruct((NP, F), jnp.float32)] * 3,
    )(x, m0, m1, c0, c1, ws, wd)


def _query_body(qr_ref, qt_ref, rel_ref, time_ref, a1_ref, a2_ref, rtb_ref,
                out_ref):
    r0 = lax.broadcasted_iota(jnp.int32, (8, NREL), 0)
    ir = lax.broadcasted_iota(jnp.int32, (8, NREL), 1)
    qv = jnp.full((8, NREL), qr_ref[3], jnp.int32)
    for j in range(3):
        qv = jnp.where(r0 == j, qr_ref[j], qv)
    oh_r = (ir == qv).astype(jnp.float32)
    t0 = lax.broadcasted_iota(jnp.int32, (8, NTS), 0)
    it = lax.broadcasted_iota(jnp.int32, (8, NTS), 1)
    tv = jnp.full((8, NTS), qt_ref[3], jnp.int32)
    for j in range(3):
        tv = jnp.where(t0 == j, qt_ref[j], tv)
    oh_t = (it == tv).astype(jnp.float32)
    rel_e = jnp.dot(oh_r, rel_ref[...], preferred_element_type=jnp.float32)
    time_e = jnp.dot(oh_t, time_ref[...], preferred_element_type=jnp.float32)
    z = (jnp.dot(rel_e, a1_ref[...], preferred_element_type=jnp.float32)
         + jnp.dot(time_e, a2_ref[...], preferred_element_type=jnp.float32)
         + rtb_ref[...])
    out_ref[...] = _lrelu(z)


def _tc_query(rel_emb, time_emb, a1, a2, rtb, q_rel, q_ts):
    return pl.pallas_call(
        _query_body,
        grid=(1,),
        in_specs=[
            pl.BlockSpec(memory_space=pltpu.SMEM),
            pl.BlockSpec(memory_space=pltpu.SMEM),
            pl.BlockSpec((NREL, H), lambda b: (0, 0)),
            pl.BlockSpec((NTS, H), lambda b: (0, 0)),
            pl.BlockSpec((H, F), lambda b: (0, 0)),
            pl.BlockSpec((H, F), lambda b: (0, 0)),
            pl.BlockSpec((1, F), lambda b: (0, 0)),
        ],
        out_specs=pl.BlockSpec((8, F), lambda b: (0, 0)),
        out_shape=jax.ShapeDtypeStruct((8, F), jnp.float32),
    )(q_rel, q_ts, rel_emb, time_emb, a1, a2, rtb)


def kernel(x_feat, node_ent, edge_index, edge_type, edge_ts, src, dst,
           q_rel, q_ts, ptr, node_emb_w, node_emb_b, rel_emb_table,
           ent_emb_table, time_emb, fc_w, fc_b, rt_w, rt_b):
    i32 = jnp.int32
    f32 = jnp.float32
    q_s = (src + ptr[:-1]).astype(i32)
    q_o = (dst + ptr[:-1]).astype(i32)

    pad_e = EP - E
    srcp = jnp.concatenate([edge_index[0].astype(i32),
                            jnp.zeros((pad_e,), i32)])
    dstp = jnp.concatenate([edge_index[1].astype(i32),
                            jnp.full((pad_e,), TRASH, i32)])
    etp = jnp.concatenate([edge_type.astype(i32), jnp.zeros((pad_e,), i32)])
    etsp = jnp.concatenate([edge_ts.astype(i32), jnp.zeros((pad_e,), i32)])
    nep = jnp.concatenate([node_ent.astype(i32), jnp.zeros((NP - N,), i32)])
    xfp = jnp.concatenate([x_feat, jnp.zeros((NP - N, x_feat.shape[1]), f32)])

    entp = jnp.concatenate(
        [ent_emb_table, jnp.zeros((N, F - H), f32)], axis=1)
    combo, ent_rows = _sc_prep(etp, etsp, nep, entp)

    a1 = rt_w[:, :H].T
    a2 = rt_w[:, H:].T
    ws = fc_w[:, :F].T
    wr = fc_w[:, F:2 * F].T
    wd = fc_w[:, 2 * F:].T
    rtb2 = rt_b.reshape(1, F)
    fcb2 = fc_b.reshape(1, F)
    nb2 = node_emb_b.reshape(1, H)

    ctab = _tc_combo(rel_emb_table, time_emb, a1, a2, rtb2, wr,
                     fcb2).reshape(NCOMBO, F)

    x, pxs, pxd, t0, tfin = _tc_node_init(
        xfp, ent_rows, node_emb_w.T, nb2, ws, wd, q_s, q_o)

    t = t0.reshape(NP)
    tnot_bit = lax.bitwise_and(t, 1)
    for _ in range(3):
        accs, cnts, cntp = _sc_edge(srcp, dstp, combo, ctab, pxs, pxd, t)
        cp = cntp.reshape(NC, NP, 16)
        x, pxs, pxd = _tc_round(x, accs[0], accs[1], cp[0], cp[1], ws, wd)
        t = 2 * (cnts[0] + cnts[1] > 0.0).astype(i32) + tnot_bit

    accs, _, cntp = _sc_edge(srcp, dstp, combo, ctab, pxs, pxd,
                             tfin.reshape(NP))
    cp = cntp.reshape(NC, NP, 16)
    x, _, _ = _tc_round(x, accs[0], accs[1], cp[0], cp[1], ws, wd)

    qrt = _tc_query(rel_emb_table, time_emb, a1, a2, rtb2, q_rel.astype(i32),
                    q_ts.astype(i32))
    return (x[:N], qrt[:4])


# speedup vs baseline: 10.1895x; 1.0009x over previous
import functools

import jax
import jax.numpy as jnp
from jax import lax
from jax.experimental import pallas as pl
from jax.experimental.pallas import tpu as pltpu
from jax.experimental.pallas import tpu_sc as plsc

N = 10000
NP = 10240
E = 320000
H = 64
F = 2 * H
NREL = 200
NTS = 365
NCOMBO = NREL * NTS

NC = 2
NS = 16
NWORK = NC * NS
CH = 128
NCHUNK = 80
PER_W = CH * NCHUNK
EP = PER_W * NWORK
TRASH = NP - 1
RPT = NP // NS
ACCW = F + 16

_mesh = plsc.VectorSubcoreMesh(
    core_axis_name="c", subcore_axis_name="s", num_cores=NC, num_subcores=NS)


def _lrelu(v):
    return jnp.maximum(v, 0.2 * v)


@functools.partial(
    pl.kernel,
    out_type=[
        jax.ShapeDtypeStruct((EP,), jnp.int32),
        jax.ShapeDtypeStruct((NP, F), jnp.float32),
    ],
    mesh=_mesh,
    scratch_types=[
        pltpu.VMEM((CH,), jnp.int32),
        pltpu.VMEM((CH,), jnp.int32),
        pltpu.VMEM((CH,), jnp.int32),
        pltpu.VMEM((80,), jnp.int32),
        pltpu.VMEM((80, F), jnp.float32),
        pltpu.SemaphoreType.DMA,
    ],
    compiler_params=pltpu.CompilerParams(needs_layout_passes=False),
)
def _sc_prep(et_hbm, ets_hbm, ne_hbm, ent_tab_hbm, combo_out, ent_out,
             et_v, ets_v, cb_v, ni_v, er_v, sem):
    cid = lax.axis_index("c")
    sid = lax.axis_index("s")
    wid = sid * NC + cid
    base = wid * PER_W

    def chunk(i):
        off = base + i * CH
        pltpu.sync_copy(et_hbm.at[pl.ds(off, CH)], et_v)
        pltpu.sync_copy(ets_hbm.at[pl.ds(off, CH)], ets_v)
        for g in range(CH // 16):
            sl = pl.ds(g * 16, 16)
            cb_v[sl] = et_v[sl] * NTS + ets_v[sl]
        pltpu.sync_copy(cb_v, combo_out.at[pl.ds(off, CH)])

    pl.loop(0, NCHUNK)(chunk)

    def nchunk(i):
        off = wid * (NP // NWORK) + i * 80
        pltpu.sync_copy(ne_hbm.at[pl.ds(off, 80)], ni_v)
        pltpu.async_copy(ent_tab_hbm.at[ni_v], er_v, sem).wait()
        pltpu.sync_copy(er_v, ent_out.at[pl.ds(off, 80)])

    pl.loop(0, (NP // NWORK) // 80)(nchunk)


def _combo_body(rel_ref, time_ref, a1_ref, a2_ref, rtb_ref, wr_ref, fcb_ref,
                out_ref):
    relc = jnp.dot(rel_ref[...].reshape(1, H), a1_ref[...],
                   preferred_element_type=jnp.float32)
    z = jnp.dot(time_ref[...], a2_ref[...],
                preferred_element_type=jnp.float32)
    z = z + relc + rtb_ref[...]
    rt = _lrelu(z)
    out = jnp.dot(rt, wr_ref[...],
                  preferred_element_type=jnp.float32) + fcb_ref[...]
    out_ref[...] = out.reshape(1, NTS, F)


def _tc_combo(rel_emb, time_emb, a1, a2, rtb, wr, fcb):
    return pl.pallas_call(
        _combo_body,
        grid=(NREL,),
        in_specs=[
            pl.BlockSpec((1, 1, H), lambda r: (r, 0, 0)),
            pl.BlockSpec((NTS, H), lambda r: (0, 0)),
            pl.BlockSpec((H, F), lambda r: (0, 0)),
            pl.BlockSpec((H, F), lambda r: (0, 0)),
            pl.BlockSpec((1, F), lambda r: (0, 0)),
            pl.BlockSpec((F, F), lambda r: (0, 0)),
            pl.BlockSpec((1, F), lambda r: (0, 0)),
        ],
        out_specs=pl.BlockSpec((1, NTS, F), lambda r: (r, 0, 0)),
        out_shape=jax.ShapeDtypeStruct((NREL, NTS, F), jnp.float32),
    )(rel_emb.reshape(NREL, 1, H), time_emb, a1, a2, rtb, wr, fcb)


_NBLK = 256
_NGRID = NP // _NBLK


def _init_body(qs_ref, qo_ref, xf_ref, ent_ref, wn_ref, nb_ref, ws_ref,
               wd_ref, x_ref, pxs_ref, pxd_ref, t0_ref, tfin_ref):
    b = pl.program_id(0)
    h = _lrelu(jnp.dot(xf_ref[...], wn_ref[...],
                       preferred_element_type=jnp.float32) + nb_ref[...])
    x = jnp.concatenate([h, ent_ref[:, pl.ds(0, H)]], axis=1)
    x_ref[...] = x
    pxs_ref[...] = jnp.dot(x, ws_ref[...], preferred_element_type=jnp.float32)
    pxd_ref[...] = jnp.dot(x, wd_ref[...], preferred_element_type=jnp.float32)
    rows = _NBLK // F
    ids = (b * _NBLK
           + lax.broadcasted_iota(jnp.int32, (rows, F), 0) * F
           + lax.broadcasted_iota(jnp.int32, (rows, F), 1))
    act = jnp.zeros((rows, F), jnp.int32)
    iso = jnp.zeros((rows, F), jnp.int32)
    for j in range(4):
        act = jnp.maximum(act, (ids == qs_ref[j]).astype(jnp.int32))
        iso = jnp.maximum(iso, (ids == qo_ref[j]).astype(jnp.int32))
    t0_ref[...] = (2 * act + (1 - iso)).reshape(1, rows, F)
    tfin_ref[...] = (2 + iso).reshape(1, rows, F)


def _tc_node_init(xf, ent_rows, wn, nb, ws, wd, q_s, q_o):
    rows = _NBLK // F
    out2 = jax.ShapeDtypeStruct((_NGRID, rows, F), jnp.int32)
    outs = [jax.ShapeDtypeStruct((NP, F), jnp.float32)] * 3 + [out2] * 2
    big = pl.BlockSpec((_NBLK, F), lambda b: (b, 0))
    tbl = pl.BlockSpec((1, rows, F), lambda b: (b, 0, 0))
    return pl.pallas_call(
        _init_body,
        grid=(_NGRID,),
        in_specs=[
            pl.BlockSpec(memory_space=pltpu.SMEM),
            pl.BlockSpec(memory_space=pltpu.SMEM),
            big,
            big,
            pl.BlockSpec((F, H), lambda b: (0, 0)),
            pl.BlockSpec((1, H), lambda b: (0, 0)),
            pl.BlockSpec((F, F), lambda b: (0, 0)),
            pl.BlockSpec((F, F), lambda b: (0, 0)),
        ],
        out_specs=[big, big, big, tbl, tbl],
        out_shape=outs,
    )(q_s, q_o, xf, ent_rows, wn, nb, ws, wd)


_CPR = NP // 8
_CPT = _CPR // NS
ECH = 32
STG = 512
NSTG = PER_W // STG


@functools.partial(
    pl.kernel,
    out_type=[
        jax.ShapeDtypeStruct((NC, NP, F), jnp.float32),
        jax.ShapeDtypeStruct((NC, NP), jnp.float32),
        jax.ShapeDtypeStruct((NC, _CPR, F), jnp.float32),
    ],
    mesh=_mesh,
    scratch_types=[
        pltpu.VMEM((STG,), jnp.int32),
        pltpu.VMEM((STG,), jnp.int32),
        pltpu.VMEM((STG,), jnp.int32),
        pltpu.VMEM((2, ECH), jnp.int32),
        pltpu.VMEM((2, ECH), jnp.int32),
        pltpu.VMEM((2 * ECH,), jnp.int32),
        pltpu.VMEM((2 * ECH,), jnp.float32),
        pltpu.VMEM((2, ECH, F), jnp.float32),
        pltpu.VMEM((2, ECH, F), jnp.float32),
        pltpu.VMEM((2, ECH, F), jnp.float32),
        pltpu.VMEM((RPT,), jnp.float32),
        pltpu.VMEM((NP,), jnp.int32),
        pltpu.VMEM_SHARED((NP, F), jnp.float32),
        pltpu.VMEM_SHARED((_CPR, F), jnp.float32),
        pltpu.SemaphoreType.DMA,
        pltpu.SemaphoreType.DMA,
        pltpu.SemaphoreType.DMA,
    ],
    compiler_params=pltpu.CompilerParams(needs_layout_passes=False),
)
def _sc_edge(src_hbm, dst_hbm, cb_hbm, ctab_hbm, pxs_hbm, pxd_hbm, t_hbm,
             acc_out, cnt_out, cntp_out,
             src_v, dst_v, ci_v, si_v, si2_v, dm_v, w_v, a_v, b_v, c_v,
             ce_v, t_v, accm_sh, accc_sh, sem1, sem2, sem3):
    cid = lax.axis_index("c")
    sid = lax.axis_index("s")
    wid = cid * NS + sid
    base = wid * PER_W
    tb = sid * RPT

    pltpu.sync_copy(t_hbm, t_v)

    zeros16 = jnp.zeros((16,), jnp.float32)

    def zrow(r):
        for s in range(F // 16):
            a_v[0, r, pl.ds(s * 16, 16)] = zeros16

    pl.loop(0, ECH)(zrow)

    def zchunk(i):
        pltpu.sync_copy(a_v.at[0], accm_sh.at[pl.ds(tb + i * ECH, ECH)])

    pl.loop(0, RPT // ECH)(zchunk)

    def zchunk2(i):
        pltpu.sync_copy(a_v.at[0].at[pl.ds(0, 16)],
                        accc_sh.at[pl.ds(sid * _CPT + i * 16, 16)])

    pl.loop(0, _CPT // 16)(zchunk2)
    plsc.subcore_barrier()

    nchk = STG // ECH

    def stage(i):
        off = base + i * STG
        e1 = pltpu.async_copy(src_hbm.at[pl.ds(off, STG)], src_v, sem2)
        e2 = pltpu.async_copy(dst_hbm.at[pl.ds(off, STG)], dst_v, sem2)
        e3 = pltpu.async_copy(cb_hbm.at[pl.ds(off, STG)], ci_v, sem2)
        e1.wait()
        e2.wait()
        e3.wait()

        def wpass(p):
            buf = p % 2
            mw = 0.0
            for g in range(ECH // 16):
                sl = pl.ds(p * ECH + g * 16, 16)
                gl = pl.ds(g * 16, 16)
                ol = pl.ds(buf * ECH + g * 16, 16)
                sv = src_v[sl]
                dv = dst_v[sl]
                ts = plsc.load_gather(t_v, [sv])
                td = plsc.load_gather(t_v, [dv])
                ok = jnp.logical_and(ts >= 2, lax.bitwise_and(td, 1) == 1)
                w = jnp.where(ok, 1.0, 0.0)
                w_v[ol] = w
                si = jnp.where(ok, dv, TRASH)
                si_v[buf, gl] = si
                si2_v[buf, gl] = lax.shift_right_logical(si, 3)
                dm_v[ol] = lax.bitwise_and(dv, 7)
                mw = jnp.maximum(mw, jnp.max(w))
            return mw

        def gxfer(p, issue):
            buf = p % 2
            ds = pl.ds(p * ECH, ECH)
            trips = [
                (pxs_hbm.at[src_v.at[ds]], a_v.at[buf]),
                (pxd_hbm.at[dst_v.at[ds]], b_v.at[buf]),
                (ctab_hbm.at[ci_v.at[ds]], c_v.at[buf]),
            ]
            for s_ref, d_ref in trips:
                if issue:
                    pltpu.async_copy(s_ref, d_ref, sem1)
                else:
                    pltpu.make_async_copy(s_ref, d_ref, sem1).wait()

        def compute(q):
            buf = q % 2

            def edge(j):
                jb = jnp.full((16,), buf * ECH + j, jnp.int32)
                wb = plsc.load_gather(w_v, [jb])

                @pl.when(jnp.max(wb) > 0.0)
                def _():
                    for s in range(F // 16):
                        sl = pl.ds(s * 16, 16)
                        v = a_v[buf, j, sl] + b_v[buf, j, sl] + c_v[buf, j, sl]
                        a_v[buf, j, sl] = _lrelu(v) * wb
                    dmb = plsc.load_gather(dm_v, [jb])
                    for k in range(8):
                        c_v[buf, j, pl.ds(k * 16, 16)] = jnp.where(
                            dmb == k, wb, 0.0)

            pl.loop(0, ECH)(edge)
            pltpu.async_copy(a_v.at[buf], accm_sh.at[si_v.at[buf]], sem3,
                             add=True)
            pltpu.async_copy(c_v.at[buf], accc_sh.at[si2_v.at[buf]], sem3,
                             add=True)

        def sdrain(q):
            buf = q % 2
            pltpu.make_async_copy(a_v.at[buf], accm_sh.at[si_v.at[buf]],
                                  sem3).wait()
            pltpu.make_async_copy(c_v.at[buf], accc_sh.at[si2_v.at[buf]],
                                  sem3).wait()

        mws = []
        for p in range(nchk + 1):
            if p < nchk:
                if p >= 2:
                    pl.when(mws[p - 2] > 0.0)(functools.partial(sdrain, p - 2))
                mw = wpass(p)
                mws.append(mw)
                pl.when(mw > 0.0)(functools.partial(gxfer, p, True))
            if p >= 1:
                q = p - 1

                def fin(q=q):
                    gxfer(q, False)
                    compute(q)

                pl.when(mws[q] > 0.0)(fin)
        for q in (nchk - 2, nchk - 1):
            pl.when(mws[q] > 0.0)(functools.partial(sdrain, q))

    pl.loop(0, NSTG)(stage)
    plsc.subcore_barrier()

    pltpu.sync_copy(accm_sh.at[pl.ds(tb, RPT)],
                    acc_out.at[cid, pl.ds(tb, RPT)])

    def cstage(i):
        pltpu.sync_copy(accc_sh.at[pl.ds(sid * _CPT + i * 16, 16)],
                        a_v.at[0].at[pl.ds(0, 16)])
        it = lax.iota(jnp.int32, 16)

        for g in range(8):
            ridx = lax.shift_right_logical(it + g * 16, 3)
            cidx = lax.bitwise_and(it, 7) * 16
            ce_v[pl.ds(i * CH + g * 16, 16)] = plsc.load_gather(
                a_v.at[0], [ridx, cidx])

        def crow(r):
            rb = jnp.full((16,), r, jnp.int32)
            for k in range(8):
                c_v[0, r, pl.ds(k * 16, 16)] = plsc.load_gather(
                    a_v.at[0], [rb, jnp.full((16,), k * 16, jnp.int32)])

        pl.loop(0, 16)(crow)
        pltpu.sync_copy(c_v.at[0].at[pl.ds(0, 16)],
                        cntp_out.at[cid, pl.ds(sid * _CPT + i * 16, 16)])

    pl.loop(0, _CPT // 16)(cstage)
    pltpu.sync_copy(ce_v, cnt_out.at[cid, pl.ds(tb, RPT)])


def _round_body(x_ref, m0_ref, m1_ref, c0_ref, c1_ref, ws_ref, wd_ref,
                xn_ref, pxs_ref, pxd_ref):
    s = m0_ref[...] + m1_ref[...]
    c = c0_ref[...] + c1_ref[...]
    cnt = jnp.max(c, axis=1, keepdims=True)
    xn = x_ref[...] + s / jnp.maximum(cnt, 1.0)
    xn_ref[...] = xn
    pxs_ref[...] = jnp.dot(xn, ws_ref[...], preferred_element_type=jnp.float32)
    pxd_ref[...] = jnp.dot(xn, wd_ref[...], preferred_element_type=jnp.float32)


def _tc_round(x, m0, m1, c0, c1, ws, wd):
    big = pl.BlockSpec((_NBLK, F), lambda b: (b, 0))
    csp = pl.BlockSpec((_NBLK, 16), lambda b: (b, 0))
    return pl.pallas_call(
        _round_body,
        grid=(_NGRID,),
        in_specs=[
            big, big, big, csp, csp,
            pl.BlockSpec((F, F), lambda b: (0, 0)),
            pl.BlockSpec((F, F), lambda b: (0, 0)),
        ],
        out_specs=[big, big, big],
        out_shape=[jax.ShapeDtypeStruct((NP, F), jnp.float32)] * 3,
    )(x, m0, m1, c0, c1, ws, wd)


def _query_body(qr_ref, qt_ref, rel_ref, time_ref, a1_ref, a2_ref, rtb_ref,
                out_ref):
    r0 = lax.broadcasted_iota(jnp.int32, (8, NREL), 0)
    ir = lax.broadcasted_iota(jnp.int32, (8, NREL), 1)
    qv = jnp.full((8, NREL), qr_ref[3], jnp.int32)
    for j in range(3):
        qv = jnp.where(r0 == j, qr_ref[j], qv)
    oh_r = (ir == qv).astype(jnp.float32)
    t0 = lax.broadcasted_iota(jnp.int32, (8, NTS), 0)
    it = lax.broadcasted_iota(jnp.int32, (8, NTS), 1)
    tv = jnp.full((8, NTS), qt_ref[3], jnp.int32)
    for j in range(3):
        tv = jnp.where(t0 == j, qt_ref[j], tv)
    oh_t = (it == tv).astype(jnp.float32)
    rel_e = jnp.dot(oh_r, rel_ref[...], preferred_element_type=jnp.float32)
    time_e = jnp.dot(oh_t, time_ref[...], preferred_element_type=jnp.float32)
    z = (jnp.dot(rel_e, a1_ref[...], preferred_element_type=jnp.float32)
         + jnp.dot(time_e, a2_ref[...], preferred_element_type=jnp.float32)
         + rtb_ref[...])
    out_ref[...] = _lrelu(z)


def _tc_query(rel_emb, time_emb, a1, a2, rtb, q_rel, q_ts):
    return pl.pallas_call(
        _query_body,
        grid=(1,),
        in_specs=[
            pl.BlockSpec(memory_space=pltpu.SMEM),
            pl.BlockSpec(memory_space=pltpu.SMEM),
            pl.BlockSpec((NREL, H), lambda b: (0, 0)),
            pl.BlockSpec((NTS, H), lambda b: (0, 0)),
            pl.BlockSpec((H, F), lambda b: (0, 0)),
            pl.BlockSpec((H, F), lambda b: (0, 0)),
            pl.BlockSpec((1, F), lambda b: (0, 0)),
        ],
        out_specs=pl.BlockSpec((8, F), lambda b: (0, 0)),
        out_shape=jax.ShapeDtypeStruct((8, F), jnp.float32),
    )(q_rel, q_ts, rel_emb, time_emb, a1, a2, rtb)


def kernel(x_feat, node_ent, edge_index, edge_type, edge_ts, src, dst,
           q_rel, q_ts, ptr, node_emb_w, node_emb_b, rel_emb_table,
           ent_emb_table, time_emb, fc_w, fc_b, rt_w, rt_b):
    i32 = jnp.int32
    f32 = jnp.float32
    q_s = (src + ptr[:-1]).astype(i32)
    q_o = (dst + ptr[:-1]).astype(i32)

    pad_e = EP - E
    srcp = jnp.concatenate([edge_index[0].astype(i32),
                            jnp.zeros((pad_e,), i32)])
    dstp = jnp.concatenate([edge_index[1].astype(i32),
                            jnp.full((pad_e,), TRASH, i32)])
    etp = jnp.concatenate([edge_type.astype(i32), jnp.zeros((pad_e,), i32)])
    etsp = jnp.concatenate([edge_ts.astype(i32), jnp.zeros((pad_e,), i32)])
    nep = jnp.concatenate([node_ent.astype(i32), jnp.zeros((NP - N,), i32)])
    xfp = jnp.concatenate([x_feat, jnp.zeros((NP - N, x_feat.shape[1]), f32)])

    entp = jnp.concatenate(
        [ent_emb_table, jnp.zeros((N, F - H), f32)], axis=1)
    combo, ent_rows = _sc_prep(etp, etsp, nep, entp)

    a1 = rt_w[:, :H].T
    a2 = rt_w[:, H:].T
    ws = fc_w[:, :F].T
    wr = fc_w[:, F:2 * F].T
    wd = fc_w[:, 2 * F:].T
    rtb2 = rt_b.reshape(1, F)
    fcb2 = fc_b.reshape(1, F)
    nb2 = node_emb_b.reshape(1, H)

    ctab = _tc_combo(rel_emb_table, time_emb, a1, a2, rtb2, wr,
                     fcb2).reshape(NCOMBO, F)

    x, pxs, pxd, t0, tfin = _tc_node_init(
        xfp, ent_rows, node_emb_w.T, nb2, ws, wd, q_s, q_o)

    t = t0.reshape(NP)
    tnot_bit = lax.bitwise_and(t, 1)
    for _ in range(3):
        accs, cnts, cntp = _sc_edge(srcp, dstp, combo, ctab, pxs, pxd, t)
        cp = cntp.reshape(NC, NP, 16)
        x, pxs, pxd = _tc_round(x, accs[0], accs[1], cp[0], cp[1], ws, wd)
        t = 2 * (cnts[0] + cnts[1] > 0.0).astype(i32) + tnot_bit

    accs, _, cntp = _sc_edge(srcp, dstp, combo, ctab, pxs, pxd,
                             tfin.reshape(NP))
    cp = cntp.reshape(NC, NP, 16)
    x, _, _ = _tc_round(x, accs[0], accs[1], cp[0], cp[1], ws, wd)

    qrt = _tc_query(rel_emb_table, time_emb, a1, a2, rtb2, q_rel.astype(i32),
                    q_ts.astype(i32))
    return (x[:N], qrt[:4])
